# baseline (speedup 1.0000x reference)
import functools
import jax
import jax.numpy as jnp
from jax import lax
from jax.experimental import pallas as pl
from jax.experimental.pallas import tpu as pltpu
from jax.experimental.pallas import tpu_sc as plsc

EPS = 1e-5
EB = 8000


def _mlp_step_body(z_ref, a_ref, c_ref, w_ref, b_ref, out_ref, s_ref, q_ref):
    i = pl.program_id(0)
    z = z_ref[...]
    h = jnp.maximum(z * a_ref[...] + c_ref[...], 0.0)
    zn = jnp.dot(h, w_ref[...], preferred_element_type=jnp.float32) + b_ref[...]
    out_ref[...] = zn
    s8 = zn.reshape(-1, 8, zn.shape[-1]).sum(0)
    q8 = (zn * zn).reshape(-1, 8, zn.shape[-1]).sum(0)

    @pl.when(i == 0)
    def _():
        s_ref[...] = s8
        q_ref[...] = q8

    @pl.when(i > 0)
    def _():
        s_ref[...] += s8
        q_ref[...] += q8


def _mlp_step(z, a, c, w, b):
    e, _ = z.shape
    wo = w.shape[1]
    eb = EB
    grid = (e // eb,)
    zn, s8, q8 = pl.pallas_call(
        _mlp_step_body,
        grid=grid,
        in_specs=[
            pl.BlockSpec((eb, z.shape[1]), lambda i: (i, 0)),
            pl.BlockSpec((1, z.shape[1]), lambda i: (0, 0)),
            pl.BlockSpec((1, z.shape[1]), lambda i: (0, 0)),
            pl.BlockSpec(w.shape, lambda i: (0, 0)),
            pl.BlockSpec((1, wo), lambda i: (0, 0)),
        ],
        out_specs=[
            pl.BlockSpec((eb, wo), lambda i: (i, 0)),
            pl.BlockSpec((8, wo), lambda i: (0, 0)),
            pl.BlockSpec((8, wo), lambda i: (0, 0)),
        ],
        out_shape=[
            jax.ShapeDtypeStruct((e, wo), jnp.float32),
            jax.ShapeDtypeStruct((8, wo), jnp.float32),
            jax.ShapeDtypeStruct((8, wo), jnp.float32),
        ],
    )(z, a, c, w, b)
    return zn, s8.sum(0), q8.sum(0)


def _edge_final_body(z_ref, a_ref, c_ref, w_ref, b_ref, out_ref):
    h = jnp.maximum(z_ref[...] * a_ref[...] + c_ref[...], 0.0)
    o = jnp.dot(h, w_ref[...], preferred_element_type=jnp.float32) + b_ref[...]
    out_ref[...] = jax.nn.sigmoid(o)


def _edge_final_body_1d(z_ref, a_ref, c_ref, w_ref, b_ref, out_ref):
    h = jnp.maximum(z_ref[...] * a_ref[...] + c_ref[...], 0.0)
    o = jnp.sum(h * w_ref[...], axis=1, keepdims=True) + b_ref[...]
    out_ref[...] = jax.nn.sigmoid(o).reshape(1, -1)


def _edge_final(z, a, c, w2, b2, eb, out1d=False):
    e, wi = z.shape
    grid = (e // eb,)
    if out1d:
        return pl.pallas_call(
            _edge_final_body_1d,
            grid=grid,
            in_specs=[
                pl.BlockSpec((eb, wi), lambda i: (i, 0)),
                pl.BlockSpec((1, wi), lambda i: (0, 0)),
                pl.BlockSpec((1, wi), lambda i: (0, 0)),
                pl.BlockSpec((1, wi), lambda i: (0, 0)),
                pl.BlockSpec((1, 1), lambda i: (0, 0)),
            ],
            out_specs=pl.BlockSpec((1, eb), lambda i: (0, i)),
            out_shape=jax.ShapeDtypeStruct((1, e), jnp.float32),
        )(z, a, c, w2.reshape(1, wi), b2)
    return pl.pallas_call(
        _edge_final_body,
        grid=grid,
        in_specs=[
            pl.BlockSpec((eb, wi), lambda i: (i, 0)),
            pl.BlockSpec((1, wi), lambda i: (0, 0)),
            pl.BlockSpec((1, wi), lambda i: (0, 0)),
            pl.BlockSpec((wi, 1), lambda i: (0, 0)),
            pl.BlockSpec((1, 1), lambda i: (0, 0)),
        ],
        out_specs=pl.BlockSpec((eb, 1), lambda i: (i, 0)),
        out_shape=jax.ShapeDtypeStruct((e, 1), jnp.float32),
    )(z, a, c, w2, b2)


def _proj1_body(x_ref, wa_ref, wb_ref, b_ref, t_ref):
    x = x_ref[...]
    wb = wb_ref[...]
    u = jnp.dot(x, wb, preferred_element_type=jnp.float32)
    p = jnp.dot(x, wa_ref[...] - wb, preferred_element_type=jnp.float32) + b_ref[...]
    t_ref[...] = jnp.concatenate([u, p], axis=1)


def _proj1(x, w1, b1):
    n, d = x.shape
    wo = w1.shape[1]
    wa, wb = w1[:d], w1[d:]
    return pl.pallas_call(
        _proj1_body,
        out_shape=jax.ShapeDtypeStruct((n, 2 * wo), jnp.float32),
    )(x, wa, wb, b1.reshape(1, wo))


def _node2_body(acc_ref, x_ref, wa_ref, wb_ref, b_ref,
                h1_ref, p_ref, u_ref, cnt_ref):
    acc = acc_ref[...]
    cnt = acc[:, 64:80].sum(axis=1, keepdims=True)
    cnt = jnp.maximum(cnt, 1.0)
    ec = acc[:, :64] / cnt
    h1 = jnp.concatenate([ec, x_ref[...]], axis=1)
    h1_ref[...] = h1
    wb = wb_ref[...]
    p_ref[...] = jnp.dot(h1, wa_ref[...] - wb, preferred_element_type=jnp.float32) + b_ref[...]
    u_ref[...] = jnp.dot(h1, wb, preferred_element_type=jnp.float32)
    cnt_ref[...] = cnt


def _node2(acc, x, w1, b1):
    n, d = x.shape
    d1 = 64 + d
    wo = w1.shape[1]
    wa, wb = w1[:d1], w1[d1:]
    return pl.pallas_call(
        _node2_body,
        out_shape=[
            jax.ShapeDtypeStruct((n, d1), jnp.float32),
            jax.ShapeDtypeStruct((n, wo), jnp.float32),
            jax.ShapeDtypeStruct((n, wo), jnp.float32),
            jax.ShapeDtypeStruct((n, 1), jnp.float32),
        ],
    )(acc, x, wa, wb, b1.reshape(1, wo))


def _node3a_body(acc_ref, cnt_ref, h1_ref, wn1_ref, bn1_ref, we_ref,
                 h2_ref, zn_ref, q_ref, s_ref, q2_ref):
    i = pl.program_id(0)
    cnt = jnp.maximum(cnt_ref[...], 1.0)
    ec = acc_ref[...] / cnt
    h2 = jnp.concatenate([ec, h1_ref[...]], axis=1)
    h2_ref[...] = h2
    zn = jnp.dot(h2, wn1_ref[...], preferred_element_type=jnp.float32) + bn1_ref[...]
    zn_ref[...] = zn
    q_ref[...] = jnp.dot(h2, we_ref[...], preferred_element_type=jnp.float32)
    s8 = zn.reshape(-1, 8, zn.shape[-1]).sum(0)
    q8 = (zn * zn).reshape(-1, 8, zn.shape[-1]).sum(0)

    @pl.when(i == 0)
    def _():
        s_ref[...] = s8
        q2_ref[...] = q8

    @pl.when(i > 0)
    def _():
        s_ref[...] += s8
        q2_ref[...] += q8


def _node3a(acc, cnt, h1, nh, we1, nb=2000):
    n, d1 = h1.shape
    w = acc.shape[-1]
    d2 = w + d1
    wq = we1.shape[1]
    grid = (n // nb,)
    h2, zn, q, s8, q8 = pl.pallas_call(
        _node3a_body,
        grid=grid,
        in_specs=[
            pl.BlockSpec((nb, w), lambda i: (i, 0)),
            pl.BlockSpec((nb, 1), lambda i: (i, 0)),
            pl.BlockSpec((nb, d1), lambda i: (i, 0)),
            pl.BlockSpec((d2, 256), lambda i: (0, 0)),
            pl.BlockSpec((1, 256), lambda i: (0, 0)),
            pl.BlockSpec((d2, wq), lambda i: (0, 0)),
        ],
        out_specs=[
            pl.BlockSpec((nb, d2), lambda i: (i, 0)),
            pl.BlockSpec((nb, 256), lambda i: (i, 0)),
            pl.BlockSpec((nb, wq), lambda i: (i, 0)),
            pl.BlockSpec((8, 256), lambda i: (0, 0)),
            pl.BlockSpec((8, 256), lambda i: (0, 0)),
        ],
        out_shape=[
            jax.ShapeDtypeStruct((n, d2), jnp.float32),
            jax.ShapeDtypeStruct((n, 256), jnp.float32),
            jax.ShapeDtypeStruct((n, wq), jnp.float32),
            jax.ShapeDtypeStruct((8, 256), jnp.float32),
            jax.ShapeDtypeStruct((8, 256), jnp.float32),
        ],
    )(acc, cnt, h1, nh["l1"]["w"], nh["l1"]["b"].reshape(1, -1), we1)
    return h2, zn, q, s8.sum(0), q8.sum(0)


_NC = 2
_NS = 16
_NW = _NC * _NS
_L = 16
_C = 80


def _sc_gather_combine_call(ta, tb, src, dst, sign, oa, ob, wout, pack=False):
    n, w = ta.shape
    e = src.shape[0]
    cc = 40 if w > 128 else _C
    nch = (e // _NW) // cc
    src3 = src.reshape(_NW, nch, cc)
    dst3 = dst.reshape(_NW, nch, cc)
    f_n = wout // _L
    zrows, zw = (cc // 2, 2 * wout) if pack else (cc, wout)
    zshape = (e // 2, 2 * wout) if pack else (e, wout)
    mesh = plsc.VectorSubcoreMesh(core_axis_name="c", subcore_axis_name="s")

    @functools.partial(
        pl.kernel,
        out_type=[jax.ShapeDtypeStruct(zshape, jnp.float32),
                  jax.ShapeDtypeStruct((_NW, 2, wout), jnp.float32)],
        mesh=mesh,
        compiler_params=pltpu.CompilerParams(use_tc_tiling_on_sc=True),
        scratch_types=[
            pltpu.VMEM((nch, cc), jnp.int32),
            pltpu.VMEM((nch, cc), jnp.int32),
            pltpu.VMEM((2, cc, w), jnp.float32),
            pltpu.VMEM((2, cc, w), jnp.float32),
            pltpu.VMEM((2, zrows, zw), jnp.float32),
            pltpu.VMEM((2, wout), jnp.float32),
            pltpu.SemaphoreType.DMA,
            pltpu.SemaphoreType.DMA,
            pltpu.SemaphoreType.DMA,
            pltpu.SemaphoreType.DMA,
            pltpu.SemaphoreType.DMA,
            pltpu.SemaphoreType.DMA,
        ])
    def k(ta_h, tb_h, s3_h, d3_h, z_h, st_h, si_v, di_v, a_v, b_v, z_v, st_v,
          ga0, ga1, gb0, gb1, zs0, zs1):
        wid = lax.axis_index("s") * _NC + lax.axis_index("c")
        row0 = wid * nch
        gsem = (ga0, ga1)
        bsem = (gb0, gb1)
        zsem = (zs0, zs1)
        pltpu.sync_copy(s3_h.at[wid], si_v)
        pltpu.sync_copy(d3_h.at[wid], di_v)
        zero = jnp.zeros((_L,), jnp.float32)
        for f in range(f_n):
            st_v[0, pl.ds(f * _L, _L)] = zero
            st_v[1, pl.ds(f * _L, _L)] = zero

        def fire(t, buf):
            pltpu.async_copy(ta_h.at[si_v.at[t]], a_v.at[buf], gsem[buf])
            pltpu.async_copy(tb_h.at[di_v.at[t]], b_v.at[buf], bsem[buf])

        def gwait(t, buf):
            pltpu.make_async_copy(ta_h.at[si_v.at[t]], a_v.at[buf],
                                  gsem[buf]).wait()
            pltpu.make_async_copy(tb_h.at[di_v.at[t]], b_v.at[buf],
                                  bsem[buf]).wait()

        def zdrain(t, buf):
            pltpu.make_async_copy(
                z_v.at[buf], z_h.at[pl.ds((row0 + t) * zrows, zrows), :],
                zsem[buf]).wait()

        npk = 2 if pack else 1

        def process(t, buf):
            gwait(t, buf)

            @pl.when(t >= 2)
            def _():
                zdrain(t, buf)

            def row(r, rc):
                s_l, q_l = rc
                ns, nq = list(rc[0]), list(rc[1])
                for j in range(npk):
                    for f in range(f_n):
                        av = a_v[buf, npk * r + j, pl.ds(oa + f * _L, _L)]
                        bv = b_v[buf, npk * r + j, pl.ds(ob + f * _L, _L)]
                        zv = av + bv if sign > 0 else av - bv
                        z_v[buf, r, pl.ds(j * wout + f * _L, _L)] = zv
                        ns[f] = ns[f] + zv
                        nq[f] = nq[f] + zv * zv
                return (tuple(ns), tuple(nq))

            z0 = (tuple(zero for _ in range(f_n)),
                  tuple(zero for _ in range(f_n)))
            s_l, q_l = lax.fori_loop(0, zrows, row, z0)
            for f in range(f_n):
                st_v[0, pl.ds(f * _L, _L)] += s_l[f]
                st_v[1, pl.ds(f * _L, _L)] += q_l[f]
            pltpu.async_copy(
                z_v.at[buf], z_h.at[pl.ds((row0 + t) * zrows, zrows), :],
                zsem[buf])

            @pl.when(t + 2 < nch)
            def _():
                fire(t + 2, buf)

        fire(0, 0)
        fire(1, 1)

        def pair(i, carry):
            process(2 * i, 0)
            process(2 * i + 1, 1)
            return carry

        lax.fori_loop(0, nch // 2, pair, 0)
        if nch % 2:
            process(nch - 1, 0)
        zdrain(nch - 2, nch % 2)
        zdrain(nch - 1, 1 - nch % 2)
        pltpu.sync_copy(st_v, st_h.at[wid])

    return k(ta, tb, src3, dst3)


def _sc_scatter_call(msg, dst, n, a, c, pack=False):
    mrows, mw = msg.shape
    e = 2 * mrows if pack else mrows
    wr = mw // 2 if pack else mw
    wt = 128
    nch = (e // _NS) // _C
    crows = _C // 2 if pack else _C
    dst3 = dst.reshape(_NS, nch, _C)
    f_n = wr // _L
    half = n // _NC
    seg = (-(-half // _NS) + _C - 1) // _C * _C
    nz = seg // _C
    tbl_rows = max(_NS * seg, half + _C)
    mesh = plsc.VectorSubcoreMesh(core_axis_name="c", subcore_axis_name="s")

    @functools.partial(
        pl.kernel,
        out_type=jax.ShapeDtypeStruct((n, wt), jnp.float32),
        mesh=mesh,
        compiler_params=pltpu.CompilerParams(use_tc_tiling_on_sc=True),
        scratch_types=[
            pltpu.VMEM((nch, _C), jnp.int32),
            pltpu.VMEM((2, crows, mw), jnp.float32),
            pltpu.VMEM((2, _C, wt), jnp.float32),
            pltpu.VMEM((1, wr), jnp.float32),
            pltpu.VMEM((1, wr), jnp.float32),
            pltpu.VMEM_SHARED((tbl_rows, wt), jnp.float32),
            pltpu.SemaphoreType.DMA,
            pltpu.SemaphoreType.DMA,
            pltpu.SemaphoreType.DMA,
            pltpu.SemaphoreType.DMA,
        ])
    def k(msg_h, d3_h, a_h, c_h, acc_h, di_v, zb_v, m_v, av_v, cv_v, table,
          ls0, ls1, ss0, ss1):
        cid = lax.axis_index("c")
        sid = lax.axis_index("s")
        row0 = sid * nch
        r0 = sid * seg
        nbase = cid * half
        lsem = (ls0, ls1)
        ssem = (ss0, ss1)
        pltpu.sync_copy(d3_h.at[sid], di_v)
        pltpu.sync_copy(a_h, av_v)
        pltpu.sync_copy(c_h, cv_v)
        zero = jnp.zeros((_L,), jnp.float32)
        dump = jnp.full((_L,), half, jnp.int32)

        def irow(r, carry):
            for j in range(_C // _L):
                v = di_v[r, pl.ds(j * _L, _L)] - nbase
                ok = (v >= 0) & (v < half)
                di_v[r, pl.ds(j * _L, _L)] = jnp.where(ok, v, dump)
            return carry

        lax.fori_loop(0, nch, irow, 0)

        def zrow(r, carry):
            for f in range(wt // _L):
                m_v[0, r, pl.ds(f * _L, _L)] = zero
                m_v[1, r, pl.ds(f * _L, _L)] = zero
            return carry

        lax.fori_loop(0, _C, zrow, 0)
        for j in range(nz):
            row = pl.multiple_of(r0 + j * _C, _C)

            @pl.when(row < tbl_rows)
            def _():
                pltpu.sync_copy(m_v.at[0], table.at[pl.ds(row, _C), :])

        if wr < wt:
            def crow(r, carry):
                m_v[0, r, pl.ds(wr, _L)] = jnp.full((_L,), 1.0 / _L,
                                                    jnp.float32)
                m_v[1, r, pl.ds(wr, _L)] = jnp.full((_L,), 1.0 / _L,
                                                    jnp.float32)
                return carry

            lax.fori_loop(0, _C, crow, 0)
        plsc.subcore_barrier()

        av_l = [av_v[0, pl.ds(f * _L, _L)] for f in range(f_n)]
        cv_l = [cv_v[0, pl.ds(f * _L, _L)] for f in range(f_n)]

        def fire(t, buf):
            pltpu.async_copy(msg_h.at[pl.ds((row0 + t) * crows, crows), :],
                             zb_v.at[buf], lsem[buf])

        def lwait(t, buf):
            pltpu.make_async_copy(msg_h.at[pl.ds((row0 + t) * crows, crows), :],
                                  zb_v.at[buf], lsem[buf]).wait()

        def sdrain(t, buf):
            pltpu.make_async_copy(m_v.at[buf], table.at[di_v.at[t]],
                                  ssem[buf]).wait()

        npk = 2 if pack else 1

        def process(t, buf):
            lwait(t, buf)

            @pl.when(t >= 2)
            def _():
                sdrain(t, buf)

            def row(r, carry):
                for j in range(npk):
                    for f in range(f_n):
                        zv = zb_v[buf, r, pl.ds(j * wr + f * _L, _L)]
                        m_v[buf, npk * r + j, pl.ds(f * _L, _L)] = jnp.maximum(
                            zv * av_l[f] + cv_l[f], 0.0)
                return carry

            lax.fori_loop(0, crows, row, 0)
            pltpu.async_copy(m_v.at[buf], table.at[di_v.at[t]],
                             ssem[buf], add=True)

            @pl.when(t + 2 < nch)
            def _():
                fire(t + 2, buf)

        fire(0, 0)
        fire(1, 1)

        def pair(i, carry):
            process(2 * i, 0)
            process(2 * i + 1, 1)
            return carry

        lax.fori_loop(0, nch // 2, pair, 0)
        if nch % 2:
            process(nch - 1, 0)
        sdrain(nch - 2, nch % 2)
        sdrain(nch - 1, 1 - nch % 2)
        plsc.subcore_barrier()

        for j in range(nz):
            row = pl.multiple_of(r0 + j * _C, _C)

            @pl.when(row + _C <= half)
            def _():
                pltpu.sync_copy(table.at[pl.ds(row, _C), :], m_v.at[0])
                pltpu.sync_copy(m_v.at[0], acc_h.at[pl.ds(nbase + row, _C), :])

            tail = half % _C
            if tail:
                @pl.when((row < half) & (row + _C > half))
                def _():
                    pltpu.sync_copy(table.at[pl.ds(row, tail), :],
                                    m_v.at[0, pl.ds(0, tail)])
                    pltpu.sync_copy(m_v.at[0, pl.ds(0, tail)],
                                    acc_h.at[pl.ds(nbase + row, tail), :])

    return k(msg, dst3, a, c)


def _gather_combine(t1, src, dst, wout):
    z, st = _sc_gather_combine_call(t1, t1, src, dst, 1, 0, wout, wout)
    return z, st[:, 0, :].sum(0), st[:, 1, :].sum(0)


def _gather_diff(q, src, dst):
    z, st = _sc_gather_combine_call(q, q, src, dst, -1, 0, 0, q.shape[1])
    return z, st[:, 0, :].sum(0), st[:, 1, :].sum(0)


def _bn_affine(bn, m, v):
    s = bn["g"] / jnp.sqrt(v + EPS)
    return (s.reshape(1, -1), (bn["b"] - m * s).reshape(1, -1))


def _edge_conv(blocks, ta, tb, oa, ob, wout, src, dst, n, pack=False):
    e = src.shape[0]

    def pk_vec(v):
        return jnp.concatenate([v, v], axis=1) if pack else v

    def pk_mat(wm):
        if not pack:
            return wm
        wz = jnp.zeros_like(wm)
        return jnp.concatenate(
            [jnp.concatenate([wm, wz], 1), jnp.concatenate([wz, wm], 1)], 0)

    def unpk(s):
        return s[:wout] + s[wout:] if pack else s

    z1, st = _sc_gather_combine_call(ta, tb, src, dst, 1, oa, ob, wout, pack)
    s1, q1 = st[:, 0, :].sum(0), st[:, 1, :].sum(0)
    m1 = s1 / e
    a1, c1 = _bn_affine(blocks[0]["bn"], m1, q1 / e - m1 * m1)
    z2, s2, q2 = _mlp_step(z1, pk_vec(a1), pk_vec(c1),
                           pk_mat(blocks[1]["lin"]["w"]),
                           pk_vec(blocks[1]["lin"]["b"].reshape(1, -1)))
    s2, q2 = unpk(s2), unpk(q2)
    m2 = s2 / e
    a2, c2 = _bn_affine(blocks[1]["bn"], m2, q2 / e - m2 * m2)
    z3, s3, q3 = _mlp_step(z2, pk_vec(a2), pk_vec(c2),
                           pk_mat(blocks[2]["lin"]["w"]),
                           pk_vec(blocks[2]["lin"]["b"].reshape(1, -1)))
    s3, q3 = unpk(s3), unpk(q3)
    m3 = s3 / e
    a3, c3 = _bn_affine(blocks[2]["bn"], m3, q3 / e - m3 * m3)
    return _sc_scatter_call(z3, dst, n, a3, c3, pack)


def kernel(x, edge_index, params):
    n = x.shape[0]
    e = edge_index.shape[1]
    src = edge_index[0]
    dst = edge_index[1]

    t1 = _proj1(x, params["ec1"][0]["lin"]["w"], params["ec1"][0]["lin"]["b"])
    acc1 = _edge_conv(params["ec1"], t1, t1, 0, 64, 64, src, dst, n, pack=True)
    h1, p2, u2, cnt = _node2(acc1, x, params["ec2"][0]["lin"]["w"],
                             params["ec2"][0]["lin"]["b"])
    acc2 = _edge_conv(params["ec2"], u2, p2, 0, 0, 128, src, dst, n)
    nh = params["node_head"]
    h2, zn, q, sn, qn = _node3a(acc2, cnt, h1, nh, params["edge_head"]["l1"]["w"])
    mn = sn / n
    an, cn = _bn_affine(nh["bn"], mn, qn / n - mn * mn)
    node_out = _edge_final(zn, an, cn, nh["l2"]["w"], nh["l2"]["b"].reshape(1, 1),
                           2000)

    ze, se, qe = _gather_diff(q, src, dst)
    me = se / e
    ae, ce = _bn_affine(params["edge_head"]["bn"], me, qe / e - me * me)
    edge_out = _edge_final(ze, ae, ce, params["edge_head"]["l2"]["w"],
                           params["edge_head"]["l2"]["b"].reshape(1, 1),
                           6400, out1d=True)
    return (node_out, edge_out.reshape(e, 1))


# speedup vs baseline: 1.1846x; 1.1846x over previous
import functools
import jax
import jax.numpy as jnp
from jax import lax
from jax.experimental import pallas as pl
from jax.experimental.pallas import tpu as pltpu
from jax.experimental.pallas import tpu_sc as plsc

EPS = 1e-5
EB = 8000


def _mlp_step_body(z_ref, a_ref, c_ref, w_ref, b_ref, out_ref, s_ref, q_ref):
    i = pl.program_id(0)
    z = z_ref[...]
    h = jnp.maximum(z * a_ref[...] + c_ref[...], 0.0)
    zn = jnp.dot(h, w_ref[...], preferred_element_type=jnp.float32) + b_ref[...]
    out_ref[...] = zn
    s8 = zn.reshape(-1, 8, zn.shape[-1]).sum(0)
    q8 = (zn * zn).reshape(-1, 8, zn.shape[-1]).sum(0)

    @pl.when(i == 0)
    def _():
        s_ref[...] = s8
        q_ref[...] = q8

    @pl.when(i > 0)
    def _():
        s_ref[...] += s8
        q_ref[...] += q8


def _mlp_step(z, a, c, w, b):
    e, _ = z.shape
    wo = w.shape[1]
    eb = EB
    grid = (e // eb,)
    zn, s8, q8 = pl.pallas_call(
        _mlp_step_body,
        grid=grid,
        in_specs=[
            pl.BlockSpec((eb, z.shape[1]), lambda i: (i, 0)),
            pl.BlockSpec((1, z.shape[1]), lambda i: (0, 0)),
            pl.BlockSpec((1, z.shape[1]), lambda i: (0, 0)),
            pl.BlockSpec(w.shape, lambda i: (0, 0)),
            pl.BlockSpec((1, wo), lambda i: (0, 0)),
        ],
        out_specs=[
            pl.BlockSpec((eb, wo), lambda i: (i, 0)),
            pl.BlockSpec((8, wo), lambda i: (0, 0)),
            pl.BlockSpec((8, wo), lambda i: (0, 0)),
        ],
        out_shape=[
            jax.ShapeDtypeStruct((e, wo), jnp.float32),
            jax.ShapeDtypeStruct((8, wo), jnp.float32),
            jax.ShapeDtypeStruct((8, wo), jnp.float32),
        ],
    )(z, a, c, w, b)
    return zn, s8.sum(0), q8.sum(0)


def _edge_final_body(z_ref, a_ref, c_ref, w_ref, b_ref, out_ref):
    h = jnp.maximum(z_ref[...] * a_ref[...] + c_ref[...], 0.0)
    o = jnp.dot(h, w_ref[...], preferred_element_type=jnp.float32) + b_ref[...]
    out_ref[...] = jax.nn.sigmoid(o)


def _edge_final_body_1d(z_ref, a_ref, c_ref, w_ref, b_ref, out_ref):
    h = jnp.maximum(z_ref[...] * a_ref[...] + c_ref[...], 0.0)
    o = jnp.dot(h, w_ref[...].reshape(-1, 1),
                preferred_element_type=jnp.float32) + b_ref[...]
    out_ref[...] = jax.nn.sigmoid(o).reshape(1, -1)


def _edge_final(z, a, c, w2, b2, eb, out1d=False):
    e, wi = z.shape
    grid = (e // eb,)
    if out1d:
        return pl.pallas_call(
            _edge_final_body_1d,
            grid=grid,
            in_specs=[
                pl.BlockSpec((eb, wi), lambda i: (i, 0)),
                pl.BlockSpec((1, wi), lambda i: (0, 0)),
                pl.BlockSpec((1, wi), lambda i: (0, 0)),
                pl.BlockSpec((1, wi), lambda i: (0, 0)),
                pl.BlockSpec((1, 1), lambda i: (0, 0)),
            ],
            out_specs=pl.BlockSpec((1, eb), lambda i: (0, i)),
            out_shape=jax.ShapeDtypeStruct((1, e), jnp.float32),
        )(z, a, c, w2.reshape(1, wi), b2)
    return pl.pallas_call(
        _edge_final_body,
        grid=grid,
        in_specs=[
            pl.BlockSpec((eb, wi), lambda i: (i, 0)),
            pl.BlockSpec((1, wi), lambda i: (0, 0)),
            pl.BlockSpec((1, wi), lambda i: (0, 0)),
            pl.BlockSpec((wi, 1), lambda i: (0, 0)),
            pl.BlockSpec((1, 1), lambda i: (0, 0)),
        ],
        out_specs=pl.BlockSpec((eb, 1), lambda i: (i, 0)),
        out_shape=jax.ShapeDtypeStruct((e, 1), jnp.float32),
    )(z, a, c, w2, b2)


def _proj1_body(x_ref, wa_ref, wb_ref, b_ref, t_ref):
    x = x_ref[...]
    wb = wb_ref[...]
    u = jnp.dot(x, wb, preferred_element_type=jnp.float32)
    p = jnp.dot(x, wa_ref[...] - wb, preferred_element_type=jnp.float32) + b_ref[...]
    t_ref[...] = jnp.concatenate([u, p], axis=1)


def _proj1(x, w1, b1):
    n, d = x.shape
    wo = w1.shape[1]
    wa, wb = w1[:d], w1[d:]
    return pl.pallas_call(
        _proj1_body,
        out_shape=jax.ShapeDtypeStruct((n, 2 * wo), jnp.float32),
    )(x, wa, wb, b1.reshape(1, wo))


def _node2_body(acc_ref, x_ref, wa_ref, wb_ref, b_ref,
                h1_ref, p_ref, u_ref, cnt_ref):
    acc = acc_ref[...]
    cnt = acc[:, 64:80].sum(axis=1, keepdims=True)
    cnt = jnp.maximum(cnt, 1.0)
    ec = acc[:, :64] / cnt
    h1 = jnp.concatenate([ec, x_ref[...]], axis=1)
    h1_ref[...] = h1
    wb = wb_ref[...]
    p_ref[...] = jnp.dot(h1, wa_ref[...] - wb, preferred_element_type=jnp.float32) + b_ref[...]
    u_ref[...] = jnp.dot(h1, wb, preferred_element_type=jnp.float32)
    cnt_ref[...] = cnt


def _node2(acc, x, w1, b1):
    n, d = x.shape
    d1 = 64 + d
    wo = w1.shape[1]
    wa, wb = w1[:d1], w1[d1:]
    return pl.pallas_call(
        _node2_body,
        out_shape=[
            jax.ShapeDtypeStruct((n, d1), jnp.float32),
            jax.ShapeDtypeStruct((n, wo), jnp.float32),
            jax.ShapeDtypeStruct((n, wo), jnp.float32),
            jax.ShapeDtypeStruct((n, 1), jnp.float32),
        ],
    )(acc, x, wa, wb, b1.reshape(1, wo))


def _node3a_body(acc_ref, cnt_ref, h1_ref, wn1_ref, bn1_ref, we_ref,
                 h2_ref, zn_ref, q_ref, s_ref, q2_ref):
    i = pl.program_id(0)
    cnt = jnp.maximum(cnt_ref[...], 1.0)
    ec = acc_ref[...] / cnt
    h2 = jnp.concatenate([ec, h1_ref[...]], axis=1)
    h2_ref[...] = h2
    zn = jnp.dot(h2, wn1_ref[...], preferred_element_type=jnp.float32) + bn1_ref[...]
    zn_ref[...] = zn
    q_ref[...] = jnp.dot(h2, we_ref[...], preferred_element_type=jnp.float32)
    s8 = zn.reshape(-1, 8, zn.shape[-1]).sum(0)
    q8 = (zn * zn).reshape(-1, 8, zn.shape[-1]).sum(0)

    @pl.when(i == 0)
    def _():
        s_ref[...] = s8
        q2_ref[...] = q8

    @pl.when(i > 0)
    def _():
        s_ref[...] += s8
        q2_ref[...] += q8


def _node3a(acc, cnt, h1, nh, we1, nb=2000):
    n, d1 = h1.shape
    w = acc.shape[-1]
    d2 = w + d1
    wq = we1.shape[1]
    grid = (n // nb,)
    h2, zn, q, s8, q8 = pl.pallas_call(
        _node3a_body,
        grid=grid,
        in_specs=[
            pl.BlockSpec((nb, w), lambda i: (i, 0)),
            pl.BlockSpec((nb, 1), lambda i: (i, 0)),
            pl.BlockSpec((nb, d1), lambda i: (i, 0)),
            pl.BlockSpec((d2, 256), lambda i: (0, 0)),
            pl.BlockSpec((1, 256), lambda i: (0, 0)),
            pl.BlockSpec((d2, wq), lambda i: (0, 0)),
        ],
        out_specs=[
            pl.BlockSpec((nb, d2), lambda i: (i, 0)),
            pl.BlockSpec((nb, 256), lambda i: (i, 0)),
            pl.BlockSpec((nb, wq), lambda i: (i, 0)),
            pl.BlockSpec((8, 256), lambda i: (0, 0)),
            pl.BlockSpec((8, 256), lambda i: (0, 0)),
        ],
        out_shape=[
            jax.ShapeDtypeStruct((n, d2), jnp.float32),
            jax.ShapeDtypeStruct((n, 256), jnp.float32),
            jax.ShapeDtypeStruct((n, wq), jnp.float32),
            jax.ShapeDtypeStruct((8, 256), jnp.float32),
            jax.ShapeDtypeStruct((8, 256), jnp.float32),
        ],
    )(acc, cnt, h1, nh["l1"]["w"], nh["l1"]["b"].reshape(1, -1), we1)
    return h2, zn, q, s8.sum(0), q8.sum(0)


_NC = 2
_NS = 16
_NW = _NC * _NS
_L = 16
_C = 80


def _sc_gather_combine_call(ta, tb, src, dst, sign, oa, ob, wout, pack=False):
    n, w = ta.shape
    e = src.shape[0]
    cc = 40 if w > 128 else _C
    nch = (e // _NW) // cc
    src3 = src.reshape(_NW, nch, cc)
    dst3 = dst.reshape(_NW, nch, cc)
    f_n = wout // _L
    zrows, zw = (cc // 2, 2 * wout) if pack else (cc, wout)
    zshape = (e // 2, 2 * wout) if pack else (e, wout)
    mesh = plsc.VectorSubcoreMesh(core_axis_name="c", subcore_axis_name="s")

    @functools.partial(
        pl.kernel,
        out_type=[jax.ShapeDtypeStruct(zshape, jnp.float32),
                  jax.ShapeDtypeStruct((_NW, 2, wout), jnp.float32)],
        mesh=mesh,
        compiler_params=pltpu.CompilerParams(use_tc_tiling_on_sc=True),
        scratch_types=[
            pltpu.VMEM((nch, cc), jnp.int32),
            pltpu.VMEM((nch, cc), jnp.int32),
            pltpu.VMEM((2, cc, w), jnp.float32),
            pltpu.VMEM((2, cc, w), jnp.float32),
            pltpu.VMEM((2, zrows, zw), jnp.float32),
            pltpu.VMEM((2, wout), jnp.float32),
            pltpu.SemaphoreType.DMA,
            pltpu.SemaphoreType.DMA,
            pltpu.SemaphoreType.DMA,
            pltpu.SemaphoreType.DMA,
            pltpu.SemaphoreType.DMA,
            pltpu.SemaphoreType.DMA,
        ])
    def k(ta_h, tb_h, s3_h, d3_h, z_h, st_h, si_v, di_v, a_v, b_v, z_v, st_v,
          ga0, ga1, gb0, gb1, zs0, zs1):
        wid = lax.axis_index("s") * _NC + lax.axis_index("c")
        row0 = wid * nch
        gsem = (ga0, ga1)
        bsem = (gb0, gb1)
        zsem = (zs0, zs1)
        pltpu.sync_copy(s3_h.at[wid], si_v)
        pltpu.sync_copy(d3_h.at[wid], di_v)
        zero = jnp.zeros((_L,), jnp.float32)
        for f in range(f_n):
            st_v[0, pl.ds(f * _L, _L)] = zero
            st_v[1, pl.ds(f * _L, _L)] = zero

        def fire(t, buf):
            pltpu.async_copy(ta_h.at[si_v.at[t]], a_v.at[buf], gsem[buf])
            pltpu.async_copy(tb_h.at[di_v.at[t]], b_v.at[buf], bsem[buf])

        def gwait(t, buf):
            pltpu.make_async_copy(ta_h.at[si_v.at[t]], a_v.at[buf],
                                  gsem[buf]).wait()
            pltpu.make_async_copy(tb_h.at[di_v.at[t]], b_v.at[buf],
                                  bsem[buf]).wait()

        def zdrain(t, buf):
            pltpu.make_async_copy(
                z_v.at[buf], z_h.at[pl.ds((row0 + t) * zrows, zrows), :],
                zsem[buf]).wait()

        npk = 2 if pack else 1

        def process(t, buf):
            gwait(t, buf)

            @pl.when(t >= 2)
            def _():
                zdrain(t, buf)

            def mkrow(j):
                def row(r, rc):
                    ns, nq = list(rc[0]), list(rc[1])
                    for f in range(f_n):
                        av = a_v[buf, j * zrows + r, pl.ds(oa + f * _L, _L)]
                        bv = b_v[buf, j * zrows + r, pl.ds(ob + f * _L, _L)]
                        zv = av + bv if sign > 0 else av - bv
                        z_v[buf, r, pl.ds(j * wout + f * _L, _L)] = zv
                        ns[f] = ns[f] + zv
                        nq[f] = nq[f] + zv * zv
                    return (tuple(ns), tuple(nq))
                return row

            z0 = (tuple(zero for _ in range(f_n)),
                  tuple(zero for _ in range(f_n)))
            for j in range(npk):
                z0 = lax.fori_loop(0, zrows, mkrow(j), z0)
            s_l, q_l = z0
            for f in range(f_n):
                st_v[0, pl.ds(f * _L, _L)] += s_l[f]
                st_v[1, pl.ds(f * _L, _L)] += q_l[f]
            pltpu.async_copy(
                z_v.at[buf], z_h.at[pl.ds((row0 + t) * zrows, zrows), :],
                zsem[buf])

            @pl.when(t + 2 < nch)
            def _():
                fire(t + 2, buf)

        fire(0, 0)
        fire(1, 1)

        def pair(i, carry):
            process(2 * i, 0)
            process(2 * i + 1, 1)
            return carry

        lax.fori_loop(0, nch // 2, pair, 0)
        if nch % 2:
            process(nch - 1, 0)
        zdrain(nch - 2, nch % 2)
        zdrain(nch - 1, 1 - nch % 2)
        pltpu.sync_copy(st_v, st_h.at[wid])

    return k(ta, tb, src3, dst3)


def _sc_scatter_call(msg, dst, n, a, c, pack=False):
    mrows, mw = msg.shape
    e = 2 * mrows if pack else mrows
    wr = mw // 2 if pack else mw
    wt = 128
    nch = (e // _NS) // _C
    crows = _C // 2 if pack else _C
    dst3 = dst.reshape(_NS, nch, _C)
    f_n = wr // _L
    half = n // _NC
    seg = (-(-half // _NS) + _C - 1) // _C * _C
    nz = seg // _C
    tbl_rows = max(_NS * seg, half + _C)
    mesh = plsc.VectorSubcoreMesh(core_axis_name="c", subcore_axis_name="s")

    @functools.partial(
        pl.kernel,
        out_type=jax.ShapeDtypeStruct((n, wt), jnp.float32),
        mesh=mesh,
        compiler_params=pltpu.CompilerParams(use_tc_tiling_on_sc=True),
        scratch_types=[
            pltpu.VMEM((nch, _C), jnp.int32),
            pltpu.VMEM((2, crows, mw), jnp.float32),
            pltpu.VMEM((2, _C, wt), jnp.float32),
            pltpu.VMEM((1, wr), jnp.float32),
            pltpu.VMEM((1, wr), jnp.float32),
            pltpu.VMEM_SHARED((tbl_rows, wt), jnp.float32),
            pltpu.SemaphoreType.DMA,
            pltpu.SemaphoreType.DMA,
            pltpu.SemaphoreType.DMA,
            pltpu.SemaphoreType.DMA,
        ])
    def k(msg_h, d3_h, a_h, c_h, acc_h, di_v, zb_v, m_v, av_v, cv_v, table,
          ls0, ls1, ss0, ss1):
        cid = lax.axis_index("c")
        sid = lax.axis_index("s")
        row0 = sid * nch
        r0 = sid * seg
        nbase = cid * half
        lsem = (ls0, ls1)
        ssem = (ss0, ss1)
        pltpu.sync_copy(d3_h.at[sid], di_v)
        pltpu.sync_copy(a_h, av_v)
        pltpu.sync_copy(c_h, cv_v)
        zero = jnp.zeros((_L,), jnp.float32)
        dump = jnp.full((_L,), half, jnp.int32)

        def irow(r, carry):
            for j in range(_C // _L):
                v = di_v[r, pl.ds(j * _L, _L)] - nbase
                ok = (v >= 0) & (v < half)
                di_v[r, pl.ds(j * _L, _L)] = jnp.where(ok, v, dump)
            return carry

        lax.fori_loop(0, nch, irow, 0)

        def zrow(r, carry):
            for f in range(wt // _L):
                m_v[0, r, pl.ds(f * _L, _L)] = zero
                m_v[1, r, pl.ds(f * _L, _L)] = zero
            return carry

        lax.fori_loop(0, _C, zrow, 0)
        for j in range(nz):
            row = pl.multiple_of(r0 + j * _C, _C)

            @pl.when(row < tbl_rows)
            def _():
                pltpu.sync_copy(m_v.at[0], table.at[pl.ds(row, _C), :])

        if wr < wt:
            def crow(r, carry):
                m_v[0, r, pl.ds(wr, _L)] = jnp.full((_L,), 1.0 / _L,
                                                    jnp.float32)
                m_v[1, r, pl.ds(wr, _L)] = jnp.full((_L,), 1.0 / _L,
                                                    jnp.float32)
                return carry

            lax.fori_loop(0, _C, crow, 0)
        plsc.subcore_barrier()

        av_l = [av_v[0, pl.ds(f * _L, _L)] for f in range(f_n)]
        cv_l = [cv_v[0, pl.ds(f * _L, _L)] for f in range(f_n)]

        def fire(t, buf):
            pltpu.async_copy(msg_h.at[pl.ds((row0 + t) * crows, crows), :],
                             zb_v.at[buf], lsem[buf])

        def lwait(t, buf):
            pltpu.make_async_copy(msg_h.at[pl.ds((row0 + t) * crows, crows), :],
                                  zb_v.at[buf], lsem[buf]).wait()

        def sdrain(t, buf):
            pltpu.make_async_copy(m_v.at[buf], table.at[di_v.at[t]],
                                  ssem[buf]).wait()

        npk = 2 if pack else 1

        def process(t, buf):
            lwait(t, buf)

            @pl.when(t >= 2)
            def _():
                sdrain(t, buf)

            for j in range(npk):
                def row(r, carry, j=j):
                    for f in range(f_n):
                        zv = zb_v[buf, r, pl.ds(j * wr + f * _L, _L)]
                        m_v[buf, j * crows + r, pl.ds(f * _L, _L)] = jnp.maximum(
                            zv * av_l[f] + cv_l[f], 0.0)
                    return carry

                lax.fori_loop(0, crows, row, 0)
            pltpu.async_copy(m_v.at[buf], table.at[di_v.at[t]],
                             ssem[buf], add=True)

            @pl.when(t + 2 < nch)
            def _():
                fire(t + 2, buf)

        fire(0, 0)
        fire(1, 1)

        def pair(i, carry):
            process(2 * i, 0)
            process(2 * i + 1, 1)
            return carry

        lax.fori_loop(0, nch // 2, pair, 0)
        if nch % 2:
            process(nch - 1, 0)
        sdrain(nch - 2, nch % 2)
        sdrain(nch - 1, 1 - nch % 2)
        plsc.subcore_barrier()

        for j in range(nz):
            row = pl.multiple_of(r0 + j * _C, _C)

            @pl.when(row + _C <= half)
            def _():
                pltpu.sync_copy(table.at[pl.ds(row, _C), :], m_v.at[0])
                pltpu.sync_copy(m_v.at[0], acc_h.at[pl.ds(nbase + row, _C), :])

            tail = half % _C
            if tail:
                @pl.when((row < half) & (row + _C > half))
                def _():
                    pltpu.sync_copy(table.at[pl.ds(row, tail), :],
                                    m_v.at[0, pl.ds(0, tail)])
                    pltpu.sync_copy(m_v.at[0, pl.ds(0, tail)],
                                    acc_h.at[pl.ds(nbase + row, tail), :])

    return k(msg, dst3, a, c)


def _gather_combine(t1, src, dst, wout):
    z, st = _sc_gather_combine_call(t1, t1, src, dst, 1, 0, wout, wout)
    return z, st[:, 0, :].sum(0), st[:, 1, :].sum(0)


def _gather_diff(q, src, dst):
    z, st = _sc_gather_combine_call(q, q, src, dst, -1, 0, 0, q.shape[1])
    return z, st[:, 0, :].sum(0), st[:, 1, :].sum(0)


def _bn_affine(bn, m, v):
    s = bn["g"] / jnp.sqrt(v + EPS)
    return (s.reshape(1, -1), (bn["b"] - m * s).reshape(1, -1))


def _edge_conv(blocks, ta, tb, oa, ob, wout, src, dst, n, pack=False):
    e = src.shape[0]

    def pk_vec(v):
        return jnp.concatenate([v, v], axis=1) if pack else v

    def pk_mat(wm):
        if not pack:
            return wm
        wz = jnp.zeros_like(wm)
        return jnp.concatenate(
            [jnp.concatenate([wm, wz], 1), jnp.concatenate([wz, wm], 1)], 0)

    def unpk(s):
        return s[:wout] + s[wout:] if pack else s

    z1, st = _sc_gather_combine_call(ta, tb, src, dst, 1, oa, ob, wout, pack)
    s1, q1 = st[:, 0, :].sum(0), st[:, 1, :].sum(0)
    m1 = s1 / e
    a1, c1 = _bn_affine(blocks[0]["bn"], m1, q1 / e - m1 * m1)
    z2, s2, q2 = _mlp_step(z1, pk_vec(a1), pk_vec(c1),
                           pk_mat(blocks[1]["lin"]["w"]),
                           pk_vec(blocks[1]["lin"]["b"].reshape(1, -1)))
    s2, q2 = unpk(s2), unpk(q2)
    m2 = s2 / e
    a2, c2 = _bn_affine(blocks[1]["bn"], m2, q2 / e - m2 * m2)
    z3, s3, q3 = _mlp_step(z2, pk_vec(a2), pk_vec(c2),
                           pk_mat(blocks[2]["lin"]["w"]),
                           pk_vec(blocks[2]["lin"]["b"].reshape(1, -1)))
    s3, q3 = unpk(s3), unpk(q3)
    m3 = s3 / e
    a3, c3 = _bn_affine(blocks[2]["bn"], m3, q3 / e - m3 * m3)
    return _sc_scatter_call(z3, dst, n, a3, c3, pack)


def kernel(x, edge_index, params):
    n = x.shape[0]
    e = edge_index.shape[1]
    src = edge_index[0]
    dst = edge_index[1]

    t1 = _proj1(x, params["ec1"][0]["lin"]["w"], params["ec1"][0]["lin"]["b"])
    acc1 = _edge_conv(params["ec1"], t1, t1, 0, 64, 64, src, dst, n, pack=True)
    h1, p2, u2, cnt = _node2(acc1, x, params["ec2"][0]["lin"]["w"],
                             params["ec2"][0]["lin"]["b"])
    acc2 = _edge_conv(params["ec2"], u2, p2, 0, 0, 128, src, dst, n)
    nh = params["node_head"]
    h2, zn, q, sn, qn = _node3a(acc2, cnt, h1, nh, params["edge_head"]["l1"]["w"])
    mn = sn / n
    an, cn = _bn_affine(nh["bn"], mn, qn / n - mn * mn)
    node_out = _edge_final(zn, an, cn, nh["l2"]["w"], nh["l2"]["b"].reshape(1, 1),
                           2000)

    ze, se, qe = _gather_diff(q, src, dst)
    me = se / e
    ae, ce = _bn_affine(params["edge_head"]["bn"], me, qe / e - me * me)
    edge_out = _edge_final(ze, ae, ce, params["edge_head"]["l2"]["w"],
                           params["edge_head"]["l2"]["b"].reshape(1, 1),
                           6400, out1d=True)
    return (node_out, edge_out.reshape(e, 1))


# speedup vs baseline: 1.1862x; 1.0013x over previous
import functools
import jax
import jax.numpy as jnp
from jax import lax
from jax.experimental import pallas as pl
from jax.experimental.pallas import tpu as pltpu
from jax.experimental.pallas import tpu_sc as plsc

EPS = 1e-5
EB = 8000


def _mlp_step_body(z_ref, a_ref, c_ref, w_ref, b_ref, out_ref, s_ref, q_ref):
    i = pl.program_id(0)
    z = z_ref[...]
    h = jnp.maximum(z * a_ref[...] + c_ref[...], 0.0)
    zn = jnp.dot(h, w_ref[...], preferred_element_type=jnp.float32) + b_ref[...]
    out_ref[...] = zn
    s8 = zn.reshape(-1, 8, zn.shape[-1]).sum(0)
    q8 = (zn * zn).reshape(-1, 8, zn.shape[-1]).sum(0)

    @pl.when(i == 0)
    def _():
        s_ref[...] = s8
        q_ref[...] = q8

    @pl.when(i > 0)
    def _():
        s_ref[...] += s8
        q_ref[...] += q8


def _mlp_step(z, a, c, w, b):
    e, _ = z.shape
    wo = w.shape[1]
    eb = EB
    grid = (e // eb,)
    zn, s8, q8 = pl.pallas_call(
        _mlp_step_body,
        grid=grid,
        in_specs=[
            pl.BlockSpec((eb, z.shape[1]), lambda i: (i, 0)),
            pl.BlockSpec((1, z.shape[1]), lambda i: (0, 0)),
            pl.BlockSpec((1, z.shape[1]), lambda i: (0, 0)),
            pl.BlockSpec(w.shape, lambda i: (0, 0)),
            pl.BlockSpec((1, wo), lambda i: (0, 0)),
        ],
        out_specs=[
            pl.BlockSpec((eb, wo), lambda i: (i, 0)),
            pl.BlockSpec((8, wo), lambda i: (0, 0)),
            pl.BlockSpec((8, wo), lambda i: (0, 0)),
        ],
        out_shape=[
            jax.ShapeDtypeStruct((e, wo), jnp.float32),
            jax.ShapeDtypeStruct((8, wo), jnp.float32),
            jax.ShapeDtypeStruct((8, wo), jnp.float32),
        ],
    )(z, a, c, w, b)
    return zn, s8.sum(0), q8.sum(0)


def _edge_final_body(z_ref, a_ref, c_ref, w_ref, b_ref, out_ref):
    h = jnp.maximum(z_ref[...] * a_ref[...] + c_ref[...], 0.0)
    o = jnp.dot(h, w_ref[...], preferred_element_type=jnp.float32) + b_ref[...]
    out_ref[...] = jax.nn.sigmoid(o)


def _edge_final_body_1d(z_ref, a_ref, c_ref, w_ref, b_ref, out_ref):
    h = jnp.maximum(z_ref[...] * a_ref[...] + c_ref[...], 0.0)
    o = jnp.dot(h, w_ref[...].reshape(-1, 1),
                preferred_element_type=jnp.float32) + b_ref[...]
    out_ref[...] = jax.nn.sigmoid(o).reshape(1, -1)


def _edge_final(z, a, c, w2, b2, eb, out1d=False):
    e, wi = z.shape
    grid = (e // eb,)
    if out1d:
        return pl.pallas_call(
            _edge_final_body_1d,
            grid=grid,
            in_specs=[
                pl.BlockSpec((eb, wi), lambda i: (i, 0)),
                pl.BlockSpec((1, wi), lambda i: (0, 0)),
                pl.BlockSpec((1, wi), lambda i: (0, 0)),
                pl.BlockSpec((1, wi), lambda i: (0, 0)),
                pl.BlockSpec((1, 1), lambda i: (0, 0)),
            ],
            out_specs=pl.BlockSpec((1, eb), lambda i: (0, i)),
            out_shape=jax.ShapeDtypeStruct((1, e), jnp.float32),
        )(z, a, c, w2.reshape(1, wi), b2)
    return pl.pallas_call(
        _edge_final_body,
        grid=grid,
        in_specs=[
            pl.BlockSpec((eb, wi), lambda i: (i, 0)),
            pl.BlockSpec((1, wi), lambda i: (0, 0)),
            pl.BlockSpec((1, wi), lambda i: (0, 0)),
            pl.BlockSpec((wi, 1), lambda i: (0, 0)),
            pl.BlockSpec((1, 1), lambda i: (0, 0)),
        ],
        out_specs=pl.BlockSpec((eb, 1), lambda i: (i, 0)),
        out_shape=jax.ShapeDtypeStruct((e, 1), jnp.float32),
    )(z, a, c, w2, b2)


def _proj1_body(x_ref, wa_ref, wb_ref, b_ref, t_ref):
    x = x_ref[...]
    wb = wb_ref[...]
    u = jnp.dot(x, wb, preferred_element_type=jnp.float32)
    p = jnp.dot(x, wa_ref[...] - wb, preferred_element_type=jnp.float32) + b_ref[...]
    t_ref[...] = jnp.concatenate([u, p], axis=1)


def _proj1(x, w1, b1):
    n, d = x.shape
    wo = w1.shape[1]
    wa, wb = w1[:d], w1[d:]
    return pl.pallas_call(
        _proj1_body,
        out_shape=jax.ShapeDtypeStruct((n, 2 * wo), jnp.float32),
    )(x, wa, wb, b1.reshape(1, wo))


def _node2_body(acc_ref, x_ref, wa_ref, wb_ref, b_ref,
                h1_ref, p_ref, u_ref, cnt_ref):
    acc = acc_ref[...]
    cnt = acc[:, 64:80].sum(axis=1, keepdims=True)
    cnt = jnp.maximum(cnt, 1.0)
    ec = acc[:, :64] / cnt
    h1 = jnp.concatenate([ec, x_ref[...]], axis=1)
    h1_ref[...] = h1
    wb = wb_ref[...]
    p_ref[...] = jnp.dot(h1, wa_ref[...] - wb, preferred_element_type=jnp.float32) + b_ref[...]
    u_ref[...] = jnp.dot(h1, wb, preferred_element_type=jnp.float32)
    cnt_ref[...] = cnt


def _node2(acc, x, w1, b1):
    n, d = x.shape
    d1 = 64 + d
    wo = w1.shape[1]
    wa, wb = w1[:d1], w1[d1:]
    return pl.pallas_call(
        _node2_body,
        out_shape=[
            jax.ShapeDtypeStruct((n, d1), jnp.float32),
            jax.ShapeDtypeStruct((n, wo), jnp.float32),
            jax.ShapeDtypeStruct((n, wo), jnp.float32),
            jax.ShapeDtypeStruct((n, 1), jnp.float32),
        ],
    )(acc, x, wa, wb, b1.reshape(1, wo))


def _node3a_body(acc_ref, cnt_ref, h1_ref, wn1_ref, bn1_ref, we_ref,
                 h2_ref, zn_ref, q_ref, s_ref, q2_ref):
    i = pl.program_id(0)
    cnt = jnp.maximum(cnt_ref[...], 1.0)
    ec = acc_ref[...] / cnt
    h2 = jnp.concatenate([ec, h1_ref[...]], axis=1)
    h2_ref[...] = h2
    zn = jnp.dot(h2, wn1_ref[...], preferred_element_type=jnp.float32) + bn1_ref[...]
    zn_ref[...] = zn
    q_ref[...] = jnp.dot(h2, we_ref[...], preferred_element_type=jnp.float32)
    s8 = zn.reshape(-1, 8, zn.shape[-1]).sum(0)
    q8 = (zn * zn).reshape(-1, 8, zn.shape[-1]).sum(0)

    @pl.when(i == 0)
    def _():
        s_ref[...] = s8
        q2_ref[...] = q8

    @pl.when(i > 0)
    def _():
        s_ref[...] += s8
        q2_ref[...] += q8


def _node3a(acc, cnt, h1, nh, we1, nb=2000):
    n, d1 = h1.shape
    w = acc.shape[-1]
    d2 = w + d1
    wq = we1.shape[1]
    grid = (n // nb,)
    h2, zn, q, s8, q8 = pl.pallas_call(
        _node3a_body,
        grid=grid,
        in_specs=[
            pl.BlockSpec((nb, w), lambda i: (i, 0)),
            pl.BlockSpec((nb, 1), lambda i: (i, 0)),
            pl.BlockSpec((nb, d1), lambda i: (i, 0)),
            pl.BlockSpec((d2, 256), lambda i: (0, 0)),
            pl.BlockSpec((1, 256), lambda i: (0, 0)),
            pl.BlockSpec((d2, wq), lambda i: (0, 0)),
        ],
        out_specs=[
            pl.BlockSpec((nb, d2), lambda i: (i, 0)),
            pl.BlockSpec((nb, 256), lambda i: (i, 0)),
            pl.BlockSpec((nb, wq), lambda i: (i, 0)),
            pl.BlockSpec((8, 256), lambda i: (0, 0)),
            pl.BlockSpec((8, 256), lambda i: (0, 0)),
        ],
        out_shape=[
            jax.ShapeDtypeStruct((n, d2), jnp.float32),
            jax.ShapeDtypeStruct((n, 256), jnp.float32),
            jax.ShapeDtypeStruct((n, wq), jnp.float32),
            jax.ShapeDtypeStruct((8, 256), jnp.float32),
            jax.ShapeDtypeStruct((8, 256), jnp.float32),
        ],
    )(acc, cnt, h1, nh["l1"]["w"], nh["l1"]["b"].reshape(1, -1), we1)
    return h2, zn, q, s8.sum(0), q8.sum(0)


_NC = 2
_NS = 16
_NW = _NC * _NS
_L = 16
_C = 80


def _sc_gather_combine_call(ta, tb, src, dst, sign, oa, ob, wout, pack=False):
    n, w = ta.shape
    e = src.shape[0]
    cc = 40 if w > 128 else _C
    nch = (e // _NW) // cc
    src3 = src.reshape(_NW, nch, cc)
    dst3 = dst.reshape(_NW, nch, cc)
    f_n = wout // _L
    zrows, zw = (cc // 2, 2 * wout) if pack else (cc, wout)
    zshape = (e // 2, 2 * wout) if pack else (e, wout)
    mesh = plsc.VectorSubcoreMesh(core_axis_name="c", subcore_axis_name="s")

    @functools.partial(
        pl.kernel,
        out_type=[jax.ShapeDtypeStruct(zshape, jnp.float32),
                  jax.ShapeDtypeStruct((_NW, 2, wout), jnp.float32)],
        mesh=mesh,
        compiler_params=pltpu.CompilerParams(use_tc_tiling_on_sc=True),
        scratch_types=[
            pltpu.VMEM((nch, cc), jnp.int32),
            pltpu.VMEM((nch, cc), jnp.int32),
            pltpu.VMEM((2, cc, w), jnp.float32),
            pltpu.VMEM((2, cc, w), jnp.float32),
            pltpu.VMEM((2, zrows, zw), jnp.float32),
            pltpu.VMEM((2, wout), jnp.float32),
            pltpu.SemaphoreType.DMA,
            pltpu.SemaphoreType.DMA,
            pltpu.SemaphoreType.DMA,
            pltpu.SemaphoreType.DMA,
            pltpu.SemaphoreType.DMA,
            pltpu.SemaphoreType.DMA,
        ])
    def k(ta_h, tb_h, s3_h, d3_h, z_h, st_h, si_v, di_v, a_v, b_v, z_v, st_v,
          ga0, ga1, gb0, gb1, zs0, zs1):
        wid = lax.axis_index("s") * _NC + lax.axis_index("c")
        row0 = wid * nch
        gsem = (ga0, ga1)
        bsem = (gb0, gb1)
        zsem = (zs0, zs1)
        pltpu.sync_copy(s3_h.at[wid], si_v)
        pltpu.sync_copy(d3_h.at[wid], di_v)
        zero = jnp.zeros((_L,), jnp.float32)
        for f in range(f_n):
            st_v[0, pl.ds(f * _L, _L)] = zero
            st_v[1, pl.ds(f * _L, _L)] = zero

        def fire(t, buf):
            pltpu.async_copy(ta_h.at[si_v.at[t]], a_v.at[buf], gsem[buf])
            pltpu.async_copy(tb_h.at[di_v.at[t]], b_v.at[buf], bsem[buf])

        def gwait(t, buf):
            pltpu.make_async_copy(ta_h.at[si_v.at[t]], a_v.at[buf],
                                  gsem[buf]).wait()
            pltpu.make_async_copy(tb_h.at[di_v.at[t]], b_v.at[buf],
                                  bsem[buf]).wait()

        def zdrain(t, buf):
            pltpu.make_async_copy(
                z_v.at[buf], z_h.at[pl.ds((row0 + t) * zrows, zrows), :],
                zsem[buf]).wait()

        npk = 2 if pack else 1

        def process(t, buf):
            gwait(t, buf)

            @pl.when(t >= 2)
            def _():
                zdrain(t, buf)

            for f0 in range(0, f_n, 8):
                fg = range(f0, min(f0 + 8, f_n))

                def mkrow(j, fg=fg):
                    def row(r, rc):
                        ns, nq = list(rc[0]), list(rc[1])
                        for i, f in enumerate(fg):
                            av = a_v[buf, j * zrows + r, pl.ds(oa + f * _L, _L)]
                            bv = b_v[buf, j * zrows + r, pl.ds(ob + f * _L, _L)]
                            zv = av + bv if sign > 0 else av - bv
                            z_v[buf, r, pl.ds(j * wout + f * _L, _L)] = zv
                            ns[i] = ns[i] + zv
                            nq[i] = nq[i] + zv * zv
                        return (tuple(ns), tuple(nq))
                    return row

                z0 = (tuple(zero for _ in fg), tuple(zero for _ in fg))
                for j in range(npk):
                    z0 = lax.fori_loop(0, zrows, mkrow(j), z0)
                s_l, q_l = z0
                for i, f in enumerate(fg):
                    st_v[0, pl.ds(f * _L, _L)] += s_l[i]
                    st_v[1, pl.ds(f * _L, _L)] += q_l[i]
            pltpu.async_copy(
                z_v.at[buf], z_h.at[pl.ds((row0 + t) * zrows, zrows), :],
                zsem[buf])

            @pl.when(t + 2 < nch)
            def _():
                fire(t + 2, buf)

        fire(0, 0)
        fire(1, 1)

        def pair(i, carry):
            process(2 * i, 0)
            process(2 * i + 1, 1)
            return carry

        lax.fori_loop(0, nch // 2, pair, 0)
        if nch % 2:
            process(nch - 1, 0)
        zdrain(nch - 2, nch % 2)
        zdrain(nch - 1, 1 - nch % 2)
        pltpu.sync_copy(st_v, st_h.at[wid])

    return k(ta, tb, src3, dst3)


def _sc_scatter_call(msg, dst, n, a, c, pack=False):
    mrows, mw = msg.shape
    e = 2 * mrows if pack else mrows
    wr = mw // 2 if pack else mw
    wt = 128
    nch = (e // _NS) // _C
    crows = _C // 2 if pack else _C
    dst3 = dst.reshape(_NS, nch, _C)
    f_n = wr // _L
    half = n // _NC
    seg = (-(-half // _NS) + _C - 1) // _C * _C
    nz = seg // _C
    tbl_rows = max(_NS * seg, half + _C)
    mesh = plsc.VectorSubcoreMesh(core_axis_name="c", subcore_axis_name="s")

    @functools.partial(
        pl.kernel,
        out_type=jax.ShapeDtypeStruct((n, wt), jnp.float32),
        mesh=mesh,
        compiler_params=pltpu.CompilerParams(use_tc_tiling_on_sc=True),
        scratch_types=[
            pltpu.VMEM((nch, _C), jnp.int32),
            pltpu.VMEM((2, crows, mw), jnp.float32),
            pltpu.VMEM((2, _C, wt), jnp.float32),
            pltpu.VMEM((1, wr), jnp.float32),
            pltpu.VMEM((1, wr), jnp.float32),
            pltpu.VMEM_SHARED((tbl_rows, wt), jnp.float32),
            pltpu.SemaphoreType.DMA,
            pltpu.SemaphoreType.DMA,
            pltpu.SemaphoreType.DMA,
            pltpu.SemaphoreType.DMA,
        ])
    def k(msg_h, d3_h, a_h, c_h, acc_h, di_v, zb_v, m_v, av_v, cv_v, table,
          ls0, ls1, ss0, ss1):
        cid = lax.axis_index("c")
        sid = lax.axis_index("s")
        row0 = sid * nch
        r0 = sid * seg
        nbase = cid * half
        lsem = (ls0, ls1)
        ssem = (ss0, ss1)
        pltpu.sync_copy(d3_h.at[sid], di_v)
        pltpu.sync_copy(a_h, av_v)
        pltpu.sync_copy(c_h, cv_v)
        zero = jnp.zeros((_L,), jnp.float32)
        dump = jnp.full((_L,), half, jnp.int32)

        def irow(r, carry):
            for j in range(_C // _L):
                v = di_v[r, pl.ds(j * _L, _L)] - nbase
                ok = (v >= 0) & (v < half)
                di_v[r, pl.ds(j * _L, _L)] = jnp.where(ok, v, dump)
            return carry

        lax.fori_loop(0, nch, irow, 0)

        def zrow(r, carry):
            for f in range(wt // _L):
                m_v[0, r, pl.ds(f * _L, _L)] = zero
                m_v[1, r, pl.ds(f * _L, _L)] = zero
            return carry

        lax.fori_loop(0, _C, zrow, 0)
        for j in range(nz):
            row = pl.multiple_of(r0 + j * _C, _C)

            @pl.when(row < tbl_rows)
            def _():
                pltpu.sync_copy(m_v.at[0], table.at[pl.ds(row, _C), :])

        if wr < wt:
            def crow(r, carry):
                m_v[0, r, pl.ds(wr, _L)] = jnp.full((_L,), 1.0 / _L,
                                                    jnp.float32)
                m_v[1, r, pl.ds(wr, _L)] = jnp.full((_L,), 1.0 / _L,
                                                    jnp.float32)
                return carry

            lax.fori_loop(0, _C, crow, 0)
        plsc.subcore_barrier()

        av_l = [av_v[0, pl.ds(f * _L, _L)] for f in range(f_n)]
        cv_l = [cv_v[0, pl.ds(f * _L, _L)] for f in range(f_n)]

        def fire(t, buf):
            pltpu.async_copy(msg_h.at[pl.ds((row0 + t) * crows, crows), :],
                             zb_v.at[buf], lsem[buf])

        def lwait(t, buf):
            pltpu.make_async_copy(msg_h.at[pl.ds((row0 + t) * crows, crows), :],
                                  zb_v.at[buf], lsem[buf]).wait()

        def sdrain(t, buf):
            pltpu.make_async_copy(m_v.at[buf], table.at[di_v.at[t]],
                                  ssem[buf]).wait()

        npk = 2 if pack else 1

        def process(t, buf):
            lwait(t, buf)

            @pl.when(t >= 2)
            def _():
                sdrain(t, buf)

            for j in range(npk):
                def row(r, carry, j=j):
                    for f in range(f_n):
                        zv = zb_v[buf, r, pl.ds(j * wr + f * _L, _L)]
                        m_v[buf, j * crows + r, pl.ds(f * _L, _L)] = jnp.maximum(
                            zv * av_l[f] + cv_l[f], 0.0)
                    return carry

                lax.fori_loop(0, crows, row, 0)
            pltpu.async_copy(m_v.at[buf], table.at[di_v.at[t]],
                             ssem[buf], add=True)

            @pl.when(t + 2 < nch)
            def _():
                fire(t + 2, buf)

        fire(0, 0)
        fire(1, 1)

        def pair(i, carry):
            process(2 * i, 0)
            process(2 * i + 1, 1)
            return carry

        lax.fori_loop(0, nch // 2, pair, 0)
        if nch % 2:
            process(nch - 1, 0)
        sdrain(nch - 2, nch % 2)
        sdrain(nch - 1, 1 - nch % 2)
        plsc.subcore_barrier()

        for j in range(nz):
            row = pl.multiple_of(r0 + j * _C, _C)

            @pl.when(row + _C <= half)
            def _():
                pltpu.sync_copy(table.at[pl.ds(row, _C), :], m_v.at[0])
                pltpu.sync_copy(m_v.at[0], acc_h.at[pl.ds(nbase + row, _C), :])

            tail = half % _C
            if tail:
                @pl.when((row < half) & (row + _C > half))
                def _():
                    pltpu.sync_copy(table.at[pl.ds(row, tail), :],
                                    m_v.at[0, pl.ds(0, tail)])
                    pltpu.sync_copy(m_v.at[0, pl.ds(0, tail)],
                                    acc_h.at[pl.ds(nbase + row, tail), :])

    return k(msg, dst3, a, c)


def _gather_combine(t1, src, dst, wout):
    z, st = _sc_gather_combine_call(t1, t1, src, dst, 1, 0, wout, wout)
    return z, st[:, 0, :].sum(0), st[:, 1, :].sum(0)


def _gather_diff(q, src, dst):
    z, st = _sc_gather_combine_call(q, q, src, dst, -1, 0, 0, q.shape[1])
    return z, st[:, 0, :].sum(0), st[:, 1, :].sum(0)


def _bn_affine(bn, m, v):
    s = bn["g"] / jnp.sqrt(v + EPS)
    return (s.reshape(1, -1), (bn["b"] - m * s).reshape(1, -1))


def _edge_conv(blocks, ta, tb, oa, ob, wout, src, dst, n, pack=False):
    e = src.shape[0]

    def pk_vec(v):
        return jnp.concatenate([v, v], axis=1) if pack else v

    def pk_mat(wm):
        if not pack:
            return wm
        wz = jnp.zeros_like(wm)
        return jnp.concatenate(
            [jnp.concatenate([wm, wz], 1), jnp.concatenate([wz, wm], 1)], 0)

    def unpk(s):
        return s[:wout] + s[wout:] if pack else s

    z1, st = _sc_gather_combine_call(ta, tb, src, dst, 1, oa, ob, wout, pack)
    s1, q1 = st[:, 0, :].sum(0), st[:, 1, :].sum(0)
    m1 = s1 / e
    a1, c1 = _bn_affine(blocks[0]["bn"], m1, q1 / e - m1 * m1)
    z2, s2, q2 = _mlp_step(z1, pk_vec(a1), pk_vec(c1),
                           pk_mat(blocks[1]["lin"]["w"]),
                           pk_vec(blocks[1]["lin"]["b"].reshape(1, -1)))
    s2, q2 = unpk(s2), unpk(q2)
    m2 = s2 / e
    a2, c2 = _bn_affine(blocks[1]["bn"], m2, q2 / e - m2 * m2)
    z3, s3, q3 = _mlp_step(z2, pk_vec(a2), pk_vec(c2),
                           pk_mat(blocks[2]["lin"]["w"]),
                           pk_vec(blocks[2]["lin"]["b"].reshape(1, -1)))
    s3, q3 = unpk(s3), unpk(q3)
    m3 = s3 / e
    a3, c3 = _bn_affine(blocks[2]["bn"], m3, q3 / e - m3 * m3)
    return _sc_scatter_call(z3, dst, n, a3, c3, pack)


def kernel(x, edge_index, params):
    n = x.shape[0]
    e = edge_index.shape[1]
    src = edge_index[0]
    dst = edge_index[1]

    t1 = _proj1(x, params["ec1"][0]["lin"]["w"], params["ec1"][0]["lin"]["b"])
    acc1 = _edge_conv(params["ec1"], t1, t1, 0, 64, 64, src, dst, n, pack=True)
    h1, p2, u2, cnt = _node2(acc1, x, params["ec2"][0]["lin"]["w"],
                             params["ec2"][0]["lin"]["b"])
    acc2 = _edge_conv(params["ec2"], u2, p2, 0, 0, 128, src, dst, n)
    nh = params["node_head"]
    h2, zn, q, sn, qn = _node3a(acc2, cnt, h1, nh, params["edge_head"]["l1"]["w"])
    mn = sn / n
    an, cn = _bn_affine(nh["bn"], mn, qn / n - mn * mn)
    node_out = _edge_final(zn, an, cn, nh["l2"]["w"], nh["l2"]["b"].reshape(1, 1),
                           2000)

    ze, se, qe = _gather_diff(q, src, dst)
    me = se / e
    ae, ce = _bn_affine(params["edge_head"]["bn"], me, qe / e - me * me)
    edge_out = _edge_final(ze, ae, ce, params["edge_head"]["l2"]["w"],
                           params["edge_head"]["l2"]["b"].reshape(1, 1),
                           6400, out1d=True)
    return (node_out, edge_out.reshape(e, 1))


# speedup vs baseline: 1.1870x; 1.0007x over previous
import functools
import jax
import jax.numpy as jnp
from jax import lax
from jax.experimental import pallas as pl
from jax.experimental.pallas import tpu as pltpu
from jax.experimental.pallas import tpu_sc as plsc

EPS = 1e-5
EB = 8000


def _mlp_step_body(z_ref, a_ref, c_ref, w_ref, b_ref, out_ref, s_ref, q_ref):
    i = pl.program_id(0)
    z = z_ref[...]
    h = jnp.maximum(z * a_ref[...] + c_ref[...], 0.0)
    zn = jnp.dot(h, w_ref[...], preferred_element_type=jnp.float32) + b_ref[...]
    out_ref[...] = zn
    s8 = zn.reshape(-1, 8, zn.shape[-1]).sum(0)
    q8 = (zn * zn).reshape(-1, 8, zn.shape[-1]).sum(0)

    @pl.when(i == 0)
    def _():
        s_ref[...] = s8
        q_ref[...] = q8

    @pl.when(i > 0)
    def _():
        s_ref[...] += s8
        q_ref[...] += q8


def _mlp_step(z, a, c, w, b):
    e, _ = z.shape
    wo = w.shape[1]
    eb = EB
    grid = (e // eb,)
    zn, s8, q8 = pl.pallas_call(
        _mlp_step_body,
        grid=grid,
        in_specs=[
            pl.BlockSpec((eb, z.shape[1]), lambda i: (i, 0)),
            pl.BlockSpec((1, z.shape[1]), lambda i: (0, 0)),
            pl.BlockSpec((1, z.shape[1]), lambda i: (0, 0)),
            pl.BlockSpec(w.shape, lambda i: (0, 0)),
            pl.BlockSpec((1, wo), lambda i: (0, 0)),
        ],
        out_specs=[
            pl.BlockSpec((eb, wo), lambda i: (i, 0)),
            pl.BlockSpec((8, wo), lambda i: (0, 0)),
            pl.BlockSpec((8, wo), lambda i: (0, 0)),
        ],
        out_shape=[
            jax.ShapeDtypeStruct((e, wo), jnp.float32),
            jax.ShapeDtypeStruct((8, wo), jnp.float32),
            jax.ShapeDtypeStruct((8, wo), jnp.float32),
        ],
    )(z, a, c, w, b)
    return zn, s8.sum(0), q8.sum(0)


def _edge_final_body(z_ref, a_ref, c_ref, w_ref, b_ref, out_ref):
    h = jnp.maximum(z_ref[...] * a_ref[...] + c_ref[...], 0.0)
    o = jnp.dot(h, w_ref[...], preferred_element_type=jnp.float32) + b_ref[...]
    out_ref[...] = jax.nn.sigmoid(o)


def _edge_final_body_1d(z_ref, a_ref, c_ref, w_ref, b_ref, out_ref):
    h = jnp.maximum(z_ref[...] * a_ref[...] + c_ref[...], 0.0)
    o = jnp.dot(h, w_ref[...], preferred_element_type=jnp.float32)
    o = o[:, 0:1] + b_ref[...]
    out_ref[...] = jax.nn.sigmoid(o).reshape(1, -1)


def _edge_final(z, a, c, w2, b2, eb, out1d=False):
    e, wi = z.shape
    grid = (e // eb,)
    if out1d:
        w2p = jnp.pad(w2, ((0, 0), (0, 127)))
        return pl.pallas_call(
            _edge_final_body_1d,
            grid=grid,
            in_specs=[
                pl.BlockSpec((eb, wi), lambda i: (i, 0)),
                pl.BlockSpec((1, wi), lambda i: (0, 0)),
                pl.BlockSpec((1, wi), lambda i: (0, 0)),
                pl.BlockSpec((wi, 128), lambda i: (0, 0)),
                pl.BlockSpec((1, 1), lambda i: (0, 0)),
            ],
            out_specs=pl.BlockSpec((1, eb), lambda i: (0, i)),
            out_shape=jax.ShapeDtypeStruct((1, e), jnp.float32),
        )(z, a, c, w2p, b2)
    return pl.pallas_call(
        _edge_final_body,
        grid=grid,
        in_specs=[
            pl.BlockSpec((eb, wi), lambda i: (i, 0)),
            pl.BlockSpec((1, wi), lambda i: (0, 0)),
            pl.BlockSpec((1, wi), lambda i: (0, 0)),
            pl.BlockSpec((wi, 1), lambda i: (0, 0)),
            pl.BlockSpec((1, 1), lambda i: (0, 0)),
        ],
        out_specs=pl.BlockSpec((eb, 1), lambda i: (i, 0)),
        out_shape=jax.ShapeDtypeStruct((e, 1), jnp.float32),
    )(z, a, c, w2, b2)


def _proj1_body(x_ref, wa_ref, wb_ref, b_ref, t_ref):
    x = x_ref[...]
    wb = wb_ref[...]
    u = jnp.dot(x, wb, preferred_element_type=jnp.float32)
    p = jnp.dot(x, wa_ref[...] - wb, preferred_element_type=jnp.float32) + b_ref[...]
    t_ref[...] = jnp.concatenate([u, p], axis=1)


def _proj1(x, w1, b1):
    n, d = x.shape
    wo = w1.shape[1]
    wa, wb = w1[:d], w1[d:]
    return pl.pallas_call(
        _proj1_body,
        out_shape=jax.ShapeDtypeStruct((n, 2 * wo), jnp.float32),
    )(x, wa, wb, b1.reshape(1, wo))


def _node2_body(acc_ref, x_ref, wa_ref, wb_ref, b_ref,
                h1_ref, p_ref, u_ref, cnt_ref):
    acc = acc_ref[...]
    cnt = acc[:, 64:80].sum(axis=1, keepdims=True)
    cnt = jnp.maximum(cnt, 1.0)
    ec = acc[:, :64] / cnt
    h1 = jnp.concatenate([ec, x_ref[...]], axis=1)
    h1_ref[...] = h1
    wb = wb_ref[...]
    p_ref[...] = jnp.dot(h1, wa_ref[...] - wb, preferred_element_type=jnp.float32) + b_ref[...]
    u_ref[...] = jnp.dot(h1, wb, preferred_element_type=jnp.float32)
    cnt_ref[...] = cnt


def _node2(acc, x, w1, b1):
    n, d = x.shape
    d1 = 64 + d
    wo = w1.shape[1]
    wa, wb = w1[:d1], w1[d1:]
    return pl.pallas_call(
        _node2_body,
        out_shape=[
            jax.ShapeDtypeStruct((n, d1), jnp.float32),
            jax.ShapeDtypeStruct((n, wo), jnp.float32),
            jax.ShapeDtypeStruct((n, wo), jnp.float32),
            jax.ShapeDtypeStruct((n, 1), jnp.float32),
        ],
    )(acc, x, wa, wb, b1.reshape(1, wo))


def _node3a_body(acc_ref, cnt_ref, h1_ref, wn1_ref, bn1_ref, we_ref,
                 h2_ref, zn_ref, q_ref, s_ref, q2_ref):
    i = pl.program_id(0)
    cnt = jnp.maximum(cnt_ref[...], 1.0)
    ec = acc_ref[...] / cnt
    h2 = jnp.concatenate([ec, h1_ref[...]], axis=1)
    h2_ref[...] = h2
    zn = jnp.dot(h2, wn1_ref[...], preferred_element_type=jnp.float32) + bn1_ref[...]
    zn_ref[...] = zn
    q_ref[...] = jnp.dot(h2, we_ref[...], preferred_element_type=jnp.float32)
    s8 = zn.reshape(-1, 8, zn.shape[-1]).sum(0)
    q8 = (zn * zn).reshape(-1, 8, zn.shape[-1]).sum(0)

    @pl.when(i == 0)
    def _():
        s_ref[...] = s8
        q2_ref[...] = q8

    @pl.when(i > 0)
    def _():
        s_ref[...] += s8
        q2_ref[...] += q8


def _node3a(acc, cnt, h1, nh, we1, nb=2000):
    n, d1 = h1.shape
    w = acc.shape[-1]
    d2 = w + d1
    wq = we1.shape[1]
    grid = (n // nb,)
    h2, zn, q, s8, q8 = pl.pallas_call(
        _node3a_body,
        grid=grid,
        in_specs=[
            pl.BlockSpec((nb, w), lambda i: (i, 0)),
            pl.BlockSpec((nb, 1), lambda i: (i, 0)),
            pl.BlockSpec((nb, d1), lambda i: (i, 0)),
            pl.BlockSpec((d2, 256), lambda i: (0, 0)),
            pl.BlockSpec((1, 256), lambda i: (0, 0)),
            pl.BlockSpec((d2, wq), lambda i: (0, 0)),
        ],
        out_specs=[
            pl.BlockSpec((nb, d2), lambda i: (i, 0)),
            pl.BlockSpec((nb, 256), lambda i: (i, 0)),
            pl.BlockSpec((nb, wq), lambda i: (i, 0)),
            pl.BlockSpec((8, 256), lambda i: (0, 0)),
            pl.BlockSpec((8, 256), lambda i: (0, 0)),
        ],
        out_shape=[
            jax.ShapeDtypeStruct((n, d2), jnp.float32),
            jax.ShapeDtypeStruct((n, 256), jnp.float32),
            jax.ShapeDtypeStruct((n, wq), jnp.float32),
            jax.ShapeDtypeStruct((8, 256), jnp.float32),
            jax.ShapeDtypeStruct((8, 256), jnp.float32),
        ],
    )(acc, cnt, h1, nh["l1"]["w"], nh["l1"]["b"].reshape(1, -1), we1)
    return h2, zn, q, s8.sum(0), q8.sum(0)


_NC = 2
_NS = 16
_NW = _NC * _NS
_L = 16
_C = 80


def _sc_gather_combine_call(ta, tb, src, dst, sign, oa, ob, wout, pack=False):
    n, w = ta.shape
    e = src.shape[0]
    cc = 40 if w > 128 else _C
    nch = (e // _NW) // cc
    src3 = src.reshape(_NW, nch, cc)
    dst3 = dst.reshape(_NW, nch, cc)
    f_n = wout // _L
    zrows, zw = (cc // 2, 2 * wout) if pack else (cc, wout)
    zshape = (e // 2, 2 * wout) if pack else (e, wout)
    mesh = plsc.VectorSubcoreMesh(core_axis_name="c", subcore_axis_name="s")

    @functools.partial(
        pl.kernel,
        out_type=[jax.ShapeDtypeStruct(zshape, jnp.float32),
                  jax.ShapeDtypeStruct((_NW, 2, wout), jnp.float32)],
        mesh=mesh,
        compiler_params=pltpu.CompilerParams(use_tc_tiling_on_sc=True),
        scratch_types=[
            pltpu.VMEM((nch, cc), jnp.int32),
            pltpu.VMEM((nch, cc), jnp.int32),
            pltpu.VMEM((2, cc, w), jnp.float32),
            pltpu.VMEM((2, cc, w), jnp.float32),
            pltpu.VMEM((2, zrows, zw), jnp.float32),
            pltpu.VMEM((2, wout), jnp.float32),
            pltpu.SemaphoreType.DMA,
            pltpu.SemaphoreType.DMA,
            pltpu.SemaphoreType.DMA,
            pltpu.SemaphoreType.DMA,
            pltpu.SemaphoreType.DMA,
            pltpu.SemaphoreType.DMA,
        ])
    def k(ta_h, tb_h, s3_h, d3_h, z_h, st_h, si_v, di_v, a_v, b_v, z_v, st_v,
          ga0, ga1, gb0, gb1, zs0, zs1):
        wid = lax.axis_index("s") * _NC + lax.axis_index("c")
        row0 = wid * nch
        gsem = (ga0, ga1)
        bsem = (gb0, gb1)
        zsem = (zs0, zs1)
        pltpu.sync_copy(s3_h.at[wid], si_v)
        pltpu.sync_copy(d3_h.at[wid], di_v)
        zero = jnp.zeros((_L,), jnp.float32)
        for f in range(f_n):
            st_v[0, pl.ds(f * _L, _L)] = zero
            st_v[1, pl.ds(f * _L, _L)] = zero

        def fire(t, buf):
            pltpu.async_copy(ta_h.at[si_v.at[t]], a_v.at[buf], gsem[buf])
            pltpu.async_copy(tb_h.at[di_v.at[t]], b_v.at[buf], bsem[buf])

        def gwait(t, buf):
            pltpu.make_async_copy(ta_h.at[si_v.at[t]], a_v.at[buf],
                                  gsem[buf]).wait()
            pltpu.make_async_copy(tb_h.at[di_v.at[t]], b_v.at[buf],
                                  bsem[buf]).wait()

        def zdrain(t, buf):
            pltpu.make_async_copy(
                z_v.at[buf], z_h.at[pl.ds((row0 + t) * zrows, zrows), :],
                zsem[buf]).wait()

        npk = 2 if pack else 1

        def process(t, buf):
            gwait(t, buf)

            @pl.when(t >= 2)
            def _():
                zdrain(t, buf)

            for f0 in range(0, f_n, 8):
                fg = range(f0, min(f0 + 8, f_n))

                def mkrow(j, fg=fg):
                    def row(r, rc):
                        ns, nq = list(rc[0]), list(rc[1])
                        for i, f in enumerate(fg):
                            av = a_v[buf, j * zrows + r, pl.ds(oa + f * _L, _L)]
                            bv = b_v[buf, j * zrows + r, pl.ds(ob + f * _L, _L)]
                            zv = av + bv if sign > 0 else av - bv
                            z_v[buf, r, pl.ds(j * wout + f * _L, _L)] = zv
                            ns[i] = ns[i] + zv
                            nq[i] = nq[i] + zv * zv
                        return (tuple(ns), tuple(nq))
                    return row

                z0 = (tuple(zero for _ in fg), tuple(zero for _ in fg))
                for j in range(npk):
                    z0 = lax.fori_loop(0, zrows, mkrow(j), z0)
                s_l, q_l = z0
                for i, f in enumerate(fg):
                    st_v[0, pl.ds(f * _L, _L)] += s_l[i]
                    st_v[1, pl.ds(f * _L, _L)] += q_l[i]
            pltpu.async_copy(
                z_v.at[buf], z_h.at[pl.ds((row0 + t) * zrows, zrows), :],
                zsem[buf])

            @pl.when(t + 2 < nch)
            def _():
                fire(t + 2, buf)

        fire(0, 0)
        fire(1, 1)

        def pair(i, carry):
            process(2 * i, 0)
            process(2 * i + 1, 1)
            return carry

        lax.fori_loop(0, nch // 2, pair, 0)
        if nch % 2:
            process(nch - 1, 0)
        zdrain(nch - 2, nch % 2)
        zdrain(nch - 1, 1 - nch % 2)
        pltpu.sync_copy(st_v, st_h.at[wid])

    return k(ta, tb, src3, dst3)


def _sc_scatter_call(msg, dst, n, a, c, pack=False):
    mrows, mw = msg.shape
    e = 2 * mrows if pack else mrows
    wr = mw // 2 if pack else mw
    wt = 128
    nch = (e // _NS) // _C
    crows = _C // 2 if pack else _C
    dst3 = dst.reshape(_NS, nch, _C)
    f_n = wr // _L
    half = n // _NC
    seg = (-(-half // _NS) + _C - 1) // _C * _C
    nz = seg // _C
    tbl_rows = max(_NS * seg, half + _C)
    mesh = plsc.VectorSubcoreMesh(core_axis_name="c", subcore_axis_name="s")

    @functools.partial(
        pl.kernel,
        out_type=jax.ShapeDtypeStruct((n, wt), jnp.float32),
        mesh=mesh,
        compiler_params=pltpu.CompilerParams(use_tc_tiling_on_sc=True),
        scratch_types=[
            pltpu.VMEM((nch, _C), jnp.int32),
            pltpu.VMEM((2, crows, mw), jnp.float32),
            pltpu.VMEM((2, _C, wt), jnp.float32),
            pltpu.VMEM((1, wr), jnp.float32),
            pltpu.VMEM((1, wr), jnp.float32),
            pltpu.VMEM_SHARED((tbl_rows, wt), jnp.float32),
            pltpu.SemaphoreType.DMA,
            pltpu.SemaphoreType.DMA,
            pltpu.SemaphoreType.DMA,
            pltpu.SemaphoreType.DMA,
        ])
    def k(msg_h, d3_h, a_h, c_h, acc_h, di_v, zb_v, m_v, av_v, cv_v, table,
          ls0, ls1, ss0, ss1):
        cid = lax.axis_index("c")
        sid = lax.axis_index("s")
        row0 = sid * nch
        r0 = sid * seg
        nbase = cid * half
        lsem = (ls0, ls1)
        ssem = (ss0, ss1)
        pltpu.sync_copy(d3_h.at[sid], di_v)
        pltpu.sync_copy(a_h, av_v)
        pltpu.sync_copy(c_h, cv_v)
        zero = jnp.zeros((_L,), jnp.float32)
        dump = jnp.full((_L,), half, jnp.int32)

        def irow(r, carry):
            for j in range(_C // _L):
                v = di_v[r, pl.ds(j * _L, _L)] - nbase
                ok = (v >= 0) & (v < half)
                di_v[r, pl.ds(j * _L, _L)] = jnp.where(ok, v, dump)
            return carry

        lax.fori_loop(0, nch, irow, 0)

        def zrow(r, carry):
            for f in range(wt // _L):
                m_v[0, r, pl.ds(f * _L, _L)] = zero
                m_v[1, r, pl.ds(f * _L, _L)] = zero
            return carry

        lax.fori_loop(0, _C, zrow, 0)
        for j in range(nz):
            row = pl.multiple_of(r0 + j * _C, _C)

            @pl.when(row < tbl_rows)
            def _():
                pltpu.sync_copy(m_v.at[0], table.at[pl.ds(row, _C), :])

        if wr < wt:
            def crow(r, carry):
                m_v[0, r, pl.ds(wr, _L)] = jnp.full((_L,), 1.0 / _L,
                                                    jnp.float32)
                m_v[1, r, pl.ds(wr, _L)] = jnp.full((_L,), 1.0 / _L,
                                                    jnp.float32)
                return carry

            lax.fori_loop(0, _C, crow, 0)
        plsc.subcore_barrier()

        av_l = [av_v[0, pl.ds(f * _L, _L)] for f in range(f_n)]
        cv_l = [cv_v[0, pl.ds(f * _L, _L)] for f in range(f_n)]

        def fire(t, buf):
            pltpu.async_copy(msg_h.at[pl.ds((row0 + t) * crows, crows), :],
                             zb_v.at[buf], lsem[buf])

        def lwait(t, buf):
            pltpu.make_async_copy(msg_h.at[pl.ds((row0 + t) * crows, crows), :],
                                  zb_v.at[buf], lsem[buf]).wait()

        def sdrain(t, buf):
            pltpu.make_async_copy(m_v.at[buf], table.at[di_v.at[t]],
                                  ssem[buf]).wait()

        npk = 2 if pack else 1

        def process(t, buf):
            lwait(t, buf)

            @pl.when(t >= 2)
            def _():
                sdrain(t, buf)

            for j in range(npk):
                def row(r, carry, j=j):
                    for f in range(f_n):
                        zv = zb_v[buf, r, pl.ds(j * wr + f * _L, _L)]
                        m_v[buf, j * crows + r, pl.ds(f * _L, _L)] = jnp.maximum(
                            zv * av_l[f] + cv_l[f], 0.0)
                    return carry

                lax.fori_loop(0, crows, row, 0)
            pltpu.async_copy(m_v.at[buf], table.at[di_v.at[t]],
                             ssem[buf], add=True)

            @pl.when(t + 2 < nch)
            def _():
                fire(t + 2, buf)

        fire(0, 0)
        fire(1, 1)

        def pair(i, carry):
            process(2 * i, 0)
            process(2 * i + 1, 1)
            return carry

        lax.fori_loop(0, nch // 2, pair, 0)
        if nch % 2:
            process(nch - 1, 0)
        sdrain(nch - 2, nch % 2)
        sdrain(nch - 1, 1 - nch % 2)
        plsc.subcore_barrier()

        for j in range(nz):
            row = pl.multiple_of(r0 + j * _C, _C)

            @pl.when(row + _C <= half)
            def _():
                pltpu.sync_copy(table.at[pl.ds(row, _C), :], m_v.at[0])
                pltpu.sync_copy(m_v.at[0], acc_h.at[pl.ds(nbase + row, _C), :])

            tail = half % _C
            if tail:
                @pl.when((row < half) & (row + _C > half))
                def _():
                    pltpu.sync_copy(table.at[pl.ds(row, tail), :],
                                    m_v.at[0, pl.ds(0, tail)])
                    pltpu.sync_copy(m_v.at[0, pl.ds(0, tail)],
                                    acc_h.at[pl.ds(nbase + row, tail), :])

    return k(msg, dst3, a, c)


def _gather_combine(t1, src, dst, wout):
    z, st = _sc_gather_combine_call(t1, t1, src, dst, 1, 0, wout, wout)
    return z, st[:, 0, :].sum(0), st[:, 1, :].sum(0)


def _gather_diff(q, src, dst):
    z, st = _sc_gather_combine_call(q, q, src, dst, -1, 0, 0, q.shape[1])
    return z, st[:, 0, :].sum(0), st[:, 1, :].sum(0)


def _bn_affine(bn, m, v):
    s = bn["g"] / jnp.sqrt(v + EPS)
    return (s.reshape(1, -1), (bn["b"] - m * s).reshape(1, -1))


def _edge_conv(blocks, ta, tb, oa, ob, wout, src, dst, n, pack=False):
    e = src.shape[0]

    def pk_vec(v):
        return jnp.concatenate([v, v], axis=1) if pack else v

    def pk_mat(wm):
        if not pack:
            return wm
        wz = jnp.zeros_like(wm)
        return jnp.concatenate(
            [jnp.concatenate([wm, wz], 1), jnp.concatenate([wz, wm], 1)], 0)

    def unpk(s):
        return s[:wout] + s[wout:] if pack else s

    z1, st = _sc_gather_combine_call(ta, tb, src, dst, 1, oa, ob, wout, pack)
    s1, q1 = st[:, 0, :].sum(0), st[:, 1, :].sum(0)
    m1 = s1 / e
    a1, c1 = _bn_affine(blocks[0]["bn"], m1, q1 / e - m1 * m1)
    z2, s2, q2 = _mlp_step(z1, pk_vec(a1), pk_vec(c1),
                           pk_mat(blocks[1]["lin"]["w"]),
                           pk_vec(blocks[1]["lin"]["b"].reshape(1, -1)))
    s2, q2 = unpk(s2), unpk(q2)
    m2 = s2 / e
    a2, c2 = _bn_affine(blocks[1]["bn"], m2, q2 / e - m2 * m2)
    z3, s3, q3 = _mlp_step(z2, pk_vec(a2), pk_vec(c2),
                           pk_mat(blocks[2]["lin"]["w"]),
                           pk_vec(blocks[2]["lin"]["b"].reshape(1, -1)))
    s3, q3 = unpk(s3), unpk(q3)
    m3 = s3 / e
    a3, c3 = _bn_affine(blocks[2]["bn"], m3, q3 / e - m3 * m3)
    return _sc_scatter_call(z3, dst, n, a3, c3, pack)


def kernel(x, edge_index, params):
    n = x.shape[0]
    e = edge_index.shape[1]
    src = edge_index[0]
    dst = edge_index[1]

    t1 = _proj1(x, params["ec1"][0]["lin"]["w"], params["ec1"][0]["lin"]["b"])
    acc1 = _edge_conv(params["ec1"], t1, t1, 0, 64, 64, src, dst, n, pack=True)
    h1, p2, u2, cnt = _node2(acc1, x, params["ec2"][0]["lin"]["w"],
                             params["ec2"][0]["lin"]["b"])
    acc2 = _edge_conv(params["ec2"], u2, p2, 0, 0, 128, src, dst, n)
    nh = params["node_head"]
    h2, zn, q, sn, qn = _node3a(acc2, cnt, h1, nh, params["edge_head"]["l1"]["w"])
    mn = sn / n
    an, cn = _bn_affine(nh["bn"], mn, qn / n - mn * mn)
    node_out = _edge_final(zn, an, cn, nh["l2"]["w"], nh["l2"]["b"].reshape(1, 1),
                           2000)

    ze, se, qe = _gather_diff(q, src, dst)
    me = se / e
    ae, ce = _bn_affine(params["edge_head"]["bn"], me, qe / e - me * me)
    edge_out = _edge_final(ze, ae, ce, params["edge_head"]["l2"]["w"],
                           params["edge_head"]["l2"]["b"].reshape(1, 1),
                           6400, out1d=True)
    return (node_out, edge_out.reshape(e, 1))


# speedup vs baseline: 1.4347x; 1.2087x over previous
import functools
import jax
import jax.numpy as jnp
from jax import lax
from jax.experimental import pallas as pl
from jax.experimental.pallas import tpu as pltpu
from jax.experimental.pallas import tpu_sc as plsc

EPS = 1e-5
EB = 8000


def _mlp_step_body(z_ref, a_ref, c_ref, w_ref, b_ref, out_ref, s_ref, q_ref):
    i = pl.program_id(0)
    z = z_ref[...]
    h = jnp.maximum(z * a_ref[...] + c_ref[...], 0.0)
    zn = jnp.dot(h, w_ref[...], preferred_element_type=jnp.float32) + b_ref[...]
    out_ref[...] = zn
    s8 = zn.reshape(-1, 8, zn.shape[-1]).sum(0)
    q8 = (zn * zn).reshape(-1, 8, zn.shape[-1]).sum(0)

    @pl.when(i == 0)
    def _():
        s_ref[...] = s8
        q_ref[...] = q8

    @pl.when(i > 0)
    def _():
        s_ref[...] += s8
        q_ref[...] += q8


def _mlp_step(z, a, c, w, b):
    e, _ = z.shape
    wo = w.shape[1]
    eb = EB
    grid = (e // eb,)
    zn, s8, q8 = pl.pallas_call(
        _mlp_step_body,
        grid=grid,
        in_specs=[
            pl.BlockSpec((eb, z.shape[1]), lambda i: (i, 0)),
            pl.BlockSpec((1, z.shape[1]), lambda i: (0, 0)),
            pl.BlockSpec((1, z.shape[1]), lambda i: (0, 0)),
            pl.BlockSpec(w.shape, lambda i: (0, 0)),
            pl.BlockSpec((1, wo), lambda i: (0, 0)),
        ],
        out_specs=[
            pl.BlockSpec((eb, wo), lambda i: (i, 0)),
            pl.BlockSpec((8, wo), lambda i: (0, 0)),
            pl.BlockSpec((8, wo), lambda i: (0, 0)),
        ],
        out_shape=[
            jax.ShapeDtypeStruct((e, wo), jnp.float32),
            jax.ShapeDtypeStruct((8, wo), jnp.float32),
            jax.ShapeDtypeStruct((8, wo), jnp.float32),
        ],
    )(z, a, c, w, b)
    return zn, s8.sum(0), q8.sum(0)


def _edge_final_body(z_ref, a_ref, c_ref, w_ref, b_ref, out_ref):
    h = jnp.maximum(z_ref[...] * a_ref[...] + c_ref[...], 0.0)
    o = jnp.dot(h, w_ref[...], preferred_element_type=jnp.float32) + b_ref[...]
    out_ref[...] = jax.nn.sigmoid(o)


def _edge_final_body_1d(z_ref, a_ref, c_ref, w_ref, b_ref, out_ref):
    h = jnp.maximum(z_ref[...] * a_ref[...] + c_ref[...], 0.0)
    o = jnp.dot(h, w_ref[...], preferred_element_type=jnp.float32)
    o = o[:, 0:1] + b_ref[...]
    out_ref[...] = jax.nn.sigmoid(o).reshape(1, -1)


def _edge_final(z, a, c, w2, b2, eb, out1d=False):
    e, wi = z.shape
    grid = (e // eb,)
    if out1d:
        w2p = jnp.pad(w2, ((0, 0), (0, 127)))
        return pl.pallas_call(
            _edge_final_body_1d,
            grid=grid,
            in_specs=[
                pl.BlockSpec((eb, wi), lambda i: (i, 0)),
                pl.BlockSpec((1, wi), lambda i: (0, 0)),
                pl.BlockSpec((1, wi), lambda i: (0, 0)),
                pl.BlockSpec((wi, 128), lambda i: (0, 0)),
                pl.BlockSpec((1, 1), lambda i: (0, 0)),
            ],
            out_specs=pl.BlockSpec((1, eb), lambda i: (0, i)),
            out_shape=jax.ShapeDtypeStruct((1, e), jnp.float32),
        )(z, a, c, w2p, b2)
    return pl.pallas_call(
        _edge_final_body,
        grid=grid,
        in_specs=[
            pl.BlockSpec((eb, wi), lambda i: (i, 0)),
            pl.BlockSpec((1, wi), lambda i: (0, 0)),
            pl.BlockSpec((1, wi), lambda i: (0, 0)),
            pl.BlockSpec((wi, 1), lambda i: (0, 0)),
            pl.BlockSpec((1, 1), lambda i: (0, 0)),
        ],
        out_specs=pl.BlockSpec((eb, 1), lambda i: (i, 0)),
        out_shape=jax.ShapeDtypeStruct((e, 1), jnp.float32),
    )(z, a, c, w2, b2)


def _proj1_body(x_ref, wa_ref, wb_ref, b_ref, t_ref):
    x = x_ref[...]
    wb = wb_ref[...]
    u = jnp.dot(x, wb, preferred_element_type=jnp.float32)
    p = jnp.dot(x, wa_ref[...] - wb, preferred_element_type=jnp.float32) + b_ref[...]
    t_ref[...] = jnp.concatenate([u, p], axis=1)


def _proj1(x, w1, b1):
    n, d = x.shape
    wo = w1.shape[1]
    wa, wb = w1[:d], w1[d:]
    return pl.pallas_call(
        _proj1_body,
        out_shape=jax.ShapeDtypeStruct((n, 2 * wo), jnp.float32),
    )(x, wa, wb, b1.reshape(1, wo))


def _node2_body(acc_ref, x_ref, wa_ref, wb_ref, b_ref,
                h1_ref, p_ref, u_ref, cnt_ref):
    acc = acc_ref[...].sum(0)
    cnt = acc[:, 64:80].sum(axis=1, keepdims=True)
    cnt = jnp.maximum(cnt, 1.0)
    ec = acc[:, :64] / cnt
    h1 = jnp.concatenate([ec, x_ref[...]], axis=1)
    h1_ref[...] = h1
    wb = wb_ref[...]
    p_ref[...] = jnp.dot(h1, wa_ref[...] - wb, preferred_element_type=jnp.float32) + b_ref[...]
    u_ref[...] = jnp.dot(h1, wb, preferred_element_type=jnp.float32)
    cnt_ref[...] = cnt


def _node2(acc, x, w1, b1):
    n, d = x.shape
    d1 = 64 + d
    wo = w1.shape[1]
    wa, wb = w1[:d1], w1[d1:]
    return pl.pallas_call(
        _node2_body,
        out_shape=[
            jax.ShapeDtypeStruct((n, d1), jnp.float32),
            jax.ShapeDtypeStruct((n, wo), jnp.float32),
            jax.ShapeDtypeStruct((n, wo), jnp.float32),
            jax.ShapeDtypeStruct((n, 1), jnp.float32),
        ],
    )(acc, x, wa, wb, b1.reshape(1, wo))


def _node3a_body(acc_ref, cnt_ref, h1_ref, wn1_ref, bn1_ref, we_ref,
                 h2_ref, zn_ref, q_ref, s_ref, q2_ref):
    i = pl.program_id(0)
    cnt = jnp.maximum(cnt_ref[...], 1.0)
    ec = acc_ref[...].sum(0) / cnt
    h2 = jnp.concatenate([ec, h1_ref[...]], axis=1)
    h2_ref[...] = h2
    zn = jnp.dot(h2, wn1_ref[...], preferred_element_type=jnp.float32) + bn1_ref[...]
    zn_ref[...] = zn
    q_ref[...] = jnp.dot(h2, we_ref[...], preferred_element_type=jnp.float32)
    s8 = zn.reshape(-1, 8, zn.shape[-1]).sum(0)
    q8 = (zn * zn).reshape(-1, 8, zn.shape[-1]).sum(0)

    @pl.when(i == 0)
    def _():
        s_ref[...] = s8
        q2_ref[...] = q8

    @pl.when(i > 0)
    def _():
        s_ref[...] += s8
        q2_ref[...] += q8


def _node3a(acc, cnt, h1, nh, we1, nb=2000):
    n, d1 = h1.shape
    w = acc.shape[-1]
    d2 = w + d1
    wq = we1.shape[1]
    grid = (n // nb,)
    h2, zn, q, s8, q8 = pl.pallas_call(
        _node3a_body,
        grid=grid,
        in_specs=[
            pl.BlockSpec((_NC, nb, w), lambda i: (0, i, 0)),
            pl.BlockSpec((nb, 1), lambda i: (i, 0)),
            pl.BlockSpec((nb, d1), lambda i: (i, 0)),
            pl.BlockSpec((d2, 256), lambda i: (0, 0)),
            pl.BlockSpec((1, 256), lambda i: (0, 0)),
            pl.BlockSpec((d2, wq), lambda i: (0, 0)),
        ],
        out_specs=[
            pl.BlockSpec((nb, d2), lambda i: (i, 0)),
            pl.BlockSpec((nb, 256), lambda i: (i, 0)),
            pl.BlockSpec((nb, wq), lambda i: (i, 0)),
            pl.BlockSpec((8, 256), lambda i: (0, 0)),
            pl.BlockSpec((8, 256), lambda i: (0, 0)),
        ],
        out_shape=[
            jax.ShapeDtypeStruct((n, d2), jnp.float32),
            jax.ShapeDtypeStruct((n, 256), jnp.float32),
            jax.ShapeDtypeStruct((n, wq), jnp.float32),
            jax.ShapeDtypeStruct((8, 256), jnp.float32),
            jax.ShapeDtypeStruct((8, 256), jnp.float32),
        ],
    )(acc, cnt, h1, nh["l1"]["w"], nh["l1"]["b"].reshape(1, -1), we1)
    return h2, zn, q, s8.sum(0), q8.sum(0)


_NC = 2
_NS = 16
_NW = _NC * _NS
_L = 16
_C = 80


def _sc_gather_combine_call(ta, tb, src, dst, sign, oa, ob, wout, pack=False):
    n, w = ta.shape
    e = src.shape[0]
    cc = 40 if w > 128 else _C
    nch = (e // _NW) // cc
    src3 = src.reshape(_NW, nch, cc)
    dst3 = dst.reshape(_NW, nch, cc)
    f_n = wout // _L
    zrows, zw = (cc // 2, 2 * wout) if pack else (cc, wout)
    zshape = (e // 2, 2 * wout) if pack else (e, wout)
    mesh = plsc.VectorSubcoreMesh(core_axis_name="c", subcore_axis_name="s")

    @functools.partial(
        pl.kernel,
        out_type=[jax.ShapeDtypeStruct(zshape, jnp.float32),
                  jax.ShapeDtypeStruct((_NW, 2, wout), jnp.float32)],
        mesh=mesh,
        compiler_params=pltpu.CompilerParams(use_tc_tiling_on_sc=True),
        scratch_types=[
            pltpu.VMEM((nch, cc), jnp.int32),
            pltpu.VMEM((nch, cc), jnp.int32),
            pltpu.VMEM((2, cc, w), jnp.float32),
            pltpu.VMEM((2, cc, w), jnp.float32),
            pltpu.VMEM((2, zrows, zw), jnp.float32),
            pltpu.VMEM((2, wout), jnp.float32),
            pltpu.SemaphoreType.DMA,
            pltpu.SemaphoreType.DMA,
            pltpu.SemaphoreType.DMA,
            pltpu.SemaphoreType.DMA,
            pltpu.SemaphoreType.DMA,
            pltpu.SemaphoreType.DMA,
        ])
    def k(ta_h, tb_h, s3_h, d3_h, z_h, st_h, si_v, di_v, a_v, b_v, z_v, st_v,
          ga0, ga1, gb0, gb1, zs0, zs1):
        wid = lax.axis_index("s") * _NC + lax.axis_index("c")
        row0 = wid * nch
        gsem = (ga0, ga1)
        bsem = (gb0, gb1)
        zsem = (zs0, zs1)
        pltpu.sync_copy(s3_h.at[wid], si_v)
        pltpu.sync_copy(d3_h.at[wid], di_v)
        zero = jnp.zeros((_L,), jnp.float32)
        for f in range(f_n):
            st_v[0, pl.ds(f * _L, _L)] = zero
            st_v[1, pl.ds(f * _L, _L)] = zero

        def fire(t, buf):
            pltpu.async_copy(ta_h.at[si_v.at[t]], a_v.at[buf], gsem[buf])
            pltpu.async_copy(tb_h.at[di_v.at[t]], b_v.at[buf], bsem[buf])

        def gwait(t, buf):
            pltpu.make_async_copy(ta_h.at[si_v.at[t]], a_v.at[buf],
                                  gsem[buf]).wait()
            pltpu.make_async_copy(tb_h.at[di_v.at[t]], b_v.at[buf],
                                  bsem[buf]).wait()

        def zdrain(t, buf):
            pltpu.make_async_copy(
                z_v.at[buf], z_h.at[pl.ds((row0 + t) * zrows, zrows), :],
                zsem[buf]).wait()

        npk = 2 if pack else 1

        def process(t, buf):
            gwait(t, buf)

            @pl.when(t >= 2)
            def _():
                zdrain(t, buf)

            for f0 in range(0, f_n, 8):
                fg = range(f0, min(f0 + 8, f_n))

                def mkrow(j, fg=fg):
                    def row(r, rc):
                        ns, nq = list(rc[0]), list(rc[1])
                        for i, f in enumerate(fg):
                            av = a_v[buf, j * zrows + r, pl.ds(oa + f * _L, _L)]
                            bv = b_v[buf, j * zrows + r, pl.ds(ob + f * _L, _L)]
                            zv = av + bv if sign > 0 else av - bv
                            z_v[buf, r, pl.ds(j * wout + f * _L, _L)] = zv
                            ns[i] = ns[i] + zv
                            nq[i] = nq[i] + zv * zv
                        return (tuple(ns), tuple(nq))
                    return row

                z0 = (tuple(zero for _ in fg), tuple(zero for _ in fg))
                for j in range(npk):
                    z0 = lax.fori_loop(0, zrows, mkrow(j), z0)
                s_l, q_l = z0
                for i, f in enumerate(fg):
                    st_v[0, pl.ds(f * _L, _L)] += s_l[i]
                    st_v[1, pl.ds(f * _L, _L)] += q_l[i]
            pltpu.async_copy(
                z_v.at[buf], z_h.at[pl.ds((row0 + t) * zrows, zrows), :],
                zsem[buf])

            @pl.when(t + 2 < nch)
            def _():
                fire(t + 2, buf)

        fire(0, 0)
        fire(1, 1)

        def pair(i, carry):
            process(2 * i, 0)
            process(2 * i + 1, 1)
            return carry

        lax.fori_loop(0, nch // 2, pair, 0)
        if nch % 2:
            process(nch - 1, 0)
        zdrain(nch - 2, nch % 2)
        zdrain(nch - 1, 1 - nch % 2)
        pltpu.sync_copy(st_v, st_h.at[wid])

    return k(ta, tb, src3, dst3)


def _sc_scatter_call(msg, dst, n, a, c, pack=False):
    mrows, mw = msg.shape
    e = 2 * mrows if pack else mrows
    wr = mw // 2 if pack else mw
    wt = 128
    nch = (e // _NW) // _C
    crows = _C // 2 if pack else _C
    dst3 = dst.reshape(_NW, nch, _C)
    f_n = wr // _L
    ncr = n // _C
    nzj = -(-ncr // _NS)
    mesh = plsc.VectorSubcoreMesh(core_axis_name="c", subcore_axis_name="s")

    @functools.partial(
        pl.kernel,
        out_type=jax.ShapeDtypeStruct((_NC, n, wt), jnp.float32),
        mesh=mesh,
        compiler_params=pltpu.CompilerParams(use_tc_tiling_on_sc=True),
        scratch_types=[
            pltpu.VMEM((4, _C), jnp.int32),
            pltpu.VMEM((2, crows, mw), jnp.float32),
            pltpu.VMEM((2, _C, wt), jnp.float32),
            pltpu.VMEM((1, wr), jnp.float32),
            pltpu.VMEM((1, wr), jnp.float32),
            pltpu.VMEM_SHARED((n, wt), jnp.float32),
            pltpu.SemaphoreType.DMA,
            pltpu.SemaphoreType.DMA,
            pltpu.SemaphoreType.DMA,
            pltpu.SemaphoreType.DMA,
            pltpu.SemaphoreType.DMA,
            pltpu.SemaphoreType.DMA,
            pltpu.SemaphoreType.DMA,
            pltpu.SemaphoreType.DMA,
        ])
    def k(msg_h, d3_h, a_h, c_h, acc_h, di_v, zb_v, m_v, av_v, cv_v, table,
          ls0, ls1, ss0, ss1, is0, is1, is2, is3):
        cid = lax.axis_index("c")
        sid = lax.axis_index("s")
        wid = sid * _NC + cid
        row0 = wid * nch
        lsem = (ls0, ls1)
        ssem = (ss0, ss1)
        isem = (is0, is1, is2, is3)
        pltpu.sync_copy(a_h, av_v)
        pltpu.sync_copy(c_h, cv_v)
        zero = jnp.zeros((_L,), jnp.float32)

        def zrow(r, carry):
            for f in range(wt // _L):
                m_v[0, r, pl.ds(f * _L, _L)] = zero
                m_v[1, r, pl.ds(f * _L, _L)] = zero
            return carry

        lax.fori_loop(0, _C, zrow, 0)
        for j in range(nzj):
            row = pl.multiple_of((sid + j * _NS) * _C, _C)

            @pl.when(row < n)
            def _():
                pltpu.sync_copy(m_v.at[0], table.at[pl.ds(row, _C), :])

        if wr < wt:
            def crow(r, carry):
                m_v[0, r, pl.ds(wr, _L)] = jnp.full((_L,), 1.0 / _L,
                                                    jnp.float32)
                m_v[1, r, pl.ds(wr, _L)] = jnp.full((_L,), 1.0 / _L,
                                                    jnp.float32)
                return carry

            lax.fori_loop(0, _C, crow, 0)
        plsc.subcore_barrier()

        av_l = [av_v[0, pl.ds(f * _L, _L)] for f in range(f_n)]
        cv_l = [cv_v[0, pl.ds(f * _L, _L)] for f in range(f_n)]

        def fire(t, buf, islot):
            pltpu.async_copy(msg_h.at[pl.ds((row0 + t) * crows, crows), :],
                             zb_v.at[buf], lsem[buf])
            pltpu.async_copy(d3_h.at[wid, pl.ds(t, 1), :],
                             di_v.at[pl.ds(islot, 1), :], isem[islot])

        def lwait(t, buf, islot):
            pltpu.make_async_copy(msg_h.at[pl.ds((row0 + t) * crows, crows), :],
                                  zb_v.at[buf], lsem[buf]).wait()
            pltpu.make_async_copy(d3_h.at[wid, pl.ds(t, 1), :],
                                  di_v.at[pl.ds(islot, 1), :],
                                  isem[islot]).wait()

        def sdrain(buf, islot):
            pltpu.make_async_copy(m_v.at[buf], table.at[di_v.at[islot]],
                                  ssem[buf]).wait()

        npk = 2 if pack else 1

        def process(t, buf, islot):
            lwait(t, buf, islot)

            @pl.when(t >= 2)
            def _():
                sdrain(buf, (islot + 2) % 4)

            for j in range(npk):
                def row(r, carry, j=j):
                    for f in range(f_n):
                        zv = zb_v[buf, r, pl.ds(j * wr + f * _L, _L)]
                        m_v[buf, j * crows + r, pl.ds(f * _L, _L)] = jnp.maximum(
                            zv * av_l[f] + cv_l[f], 0.0)
                    return carry

                lax.fori_loop(0, crows, row, 0)
            pltpu.async_copy(m_v.at[buf], table.at[di_v.at[islot]],
                             ssem[buf], add=True)

            @pl.when(t + 2 < nch)
            def _():
                fire(t + 2, buf, (islot + 2) % 4)

        fire(0, 0, 0)
        fire(1, 1, 1)

        def quad(i, carry):
            process(4 * i, 0, 0)
            process(4 * i + 1, 1, 1)
            process(4 * i + 2, 0, 2)
            process(4 * i + 3, 1, 3)
            return carry

        lax.fori_loop(0, nch // 4, quad, 0)
        for j in range(nch % 4):
            t = nch - nch % 4 + j
            process(t, t % 2, t % 4)
        sdrain((nch - 2) % 2, (nch - 2) % 4)
        sdrain((nch - 1) % 2, (nch - 1) % 4)
        plsc.subcore_barrier()

        for j in range(nzj):
            row = pl.multiple_of((sid + j * _NS) * _C, _C)

            @pl.when(row < n)
            def _():
                pltpu.sync_copy(table.at[pl.ds(row, _C), :], m_v.at[0])
                pltpu.sync_copy(m_v.at[0], acc_h.at[cid, pl.ds(row, _C), :])

    return k(msg, dst3, a, c)


def _gather_combine(t1, src, dst, wout):
    z, st = _sc_gather_combine_call(t1, t1, src, dst, 1, 0, wout, wout)
    return z, st[:, 0, :].sum(0), st[:, 1, :].sum(0)


def _gather_diff(q, src, dst):
    z, st = _sc_gather_combine_call(q, q, src, dst, -1, 0, 0, q.shape[1])
    return z, st[:, 0, :].sum(0), st[:, 1, :].sum(0)


def _bn_affine(bn, m, v):
    s = bn["g"] / jnp.sqrt(v + EPS)
    return (s.reshape(1, -1), (bn["b"] - m * s).reshape(1, -1))


def _edge_conv(blocks, ta, tb, oa, ob, wout, src, dst, n, pack=False):
    e = src.shape[0]

    def pk_vec(v):
        return jnp.concatenate([v, v], axis=1) if pack else v

    def pk_mat(wm):
        if not pack:
            return wm
        wz = jnp.zeros_like(wm)
        return jnp.concatenate(
            [jnp.concatenate([wm, wz], 1), jnp.concatenate([wz, wm], 1)], 0)

    def unpk(s):
        return s[:wout] + s[wout:] if pack else s

    z1, st = _sc_gather_combine_call(ta, tb, src, dst, 1, oa, ob, wout, pack)
    s1, q1 = st[:, 0, :].sum(0), st[:, 1, :].sum(0)
    m1 = s1 / e
    a1, c1 = _bn_affine(blocks[0]["bn"], m1, q1 / e - m1 * m1)
    z2, s2, q2 = _mlp_step(z1, pk_vec(a1), pk_vec(c1),
                           pk_mat(blocks[1]["lin"]["w"]),
                           pk_vec(blocks[1]["lin"]["b"].reshape(1, -1)))
    s2, q2 = unpk(s2), unpk(q2)
    m2 = s2 / e
    a2, c2 = _bn_affine(blocks[1]["bn"], m2, q2 / e - m2 * m2)
    z3, s3, q3 = _mlp_step(z2, pk_vec(a2), pk_vec(c2),
                           pk_mat(blocks[2]["lin"]["w"]),
                           pk_vec(blocks[2]["lin"]["b"].reshape(1, -1)))
    s3, q3 = unpk(s3), unpk(q3)
    m3 = s3 / e
    a3, c3 = _bn_affine(blocks[2]["bn"], m3, q3 / e - m3 * m3)
    return _sc_scatter_call(z3, dst, n, a3, c3, pack)


def kernel(x, edge_index, params):
    n = x.shape[0]
    e = edge_index.shape[1]
    src = edge_index[0]
    dst = edge_index[1]

    t1 = _proj1(x, params["ec1"][0]["lin"]["w"], params["ec1"][0]["lin"]["b"])
    acc1 = _edge_conv(params["ec1"], t1, t1, 0, 64, 64, src, dst, n, pack=True)
    h1, p2, u2, cnt = _node2(acc1, x, params["ec2"][0]["lin"]["w"],
                             params["ec2"][0]["lin"]["b"])
    acc2 = _edge_conv(params["ec2"], u2, p2, 0, 0, 128, src, dst, n)
    nh = params["node_head"]
    h2, zn, q, sn, qn = _node3a(acc2, cnt, h1, nh, params["edge_head"]["l1"]["w"])
    mn = sn / n
    an, cn = _bn_affine(nh["bn"], mn, qn / n - mn * mn)
    node_out = _edge_final(zn, an, cn, nh["l2"]["w"], nh["l2"]["b"].reshape(1, 1),
                           2000)

    ze, se, qe = _gather_diff(q, src, dst)
    me = se / e
    ae, ce = _bn_affine(params["edge_head"]["bn"], me, qe / e - me * me)
    edge_out = _edge_final(ze, ae, ce, params["edge_head"]["l2"]["w"],
                           params["edge_head"]["l2"]["b"].reshape(1, 1),
                           6400, out1d=True)
    return (node_out, edge_out.reshape(e, 1))


# speedup vs baseline: 1.4874x; 1.0367x over previous
import functools
import jax
import jax.numpy as jnp
from jax import lax
from jax.experimental import pallas as pl
from jax.experimental.pallas import tpu as pltpu
from jax.experimental.pallas import tpu_sc as plsc

EPS = 1e-5
EB = 8000


def _mlp_step_body(z_ref, a_ref, c_ref, w_ref, b_ref, out_ref, s_ref, q_ref):
    i = pl.program_id(0)
    z = z_ref[...].astype(jnp.float32)
    h = jnp.maximum(z * a_ref[...] + c_ref[...], 0.0)
    zn = jnp.dot(h, w_ref[...], preferred_element_type=jnp.float32) + b_ref[...]
    out_ref[...] = zn.astype(out_ref.dtype)
    zn = zn.astype(out_ref.dtype).astype(jnp.float32)
    s8 = zn.reshape(-1, 8, zn.shape[-1]).sum(0)
    q8 = (zn * zn).reshape(-1, 8, zn.shape[-1]).sum(0)

    @pl.when(i == 0)
    def _():
        s_ref[...] = s8
        q_ref[...] = q8

    @pl.when(i > 0)
    def _():
        s_ref[...] += s8
        q_ref[...] += q8


def _mlp_step(z, a, c, w, b, out_dtype=jnp.float32):
    e, _ = z.shape
    wo = w.shape[1]
    eb = EB
    grid = (e // eb,)
    zn, s8, q8 = pl.pallas_call(
        _mlp_step_body,
        grid=grid,
        in_specs=[
            pl.BlockSpec((eb, z.shape[1]), lambda i: (i, 0)),
            pl.BlockSpec((1, z.shape[1]), lambda i: (0, 0)),
            pl.BlockSpec((1, z.shape[1]), lambda i: (0, 0)),
            pl.BlockSpec(w.shape, lambda i: (0, 0)),
            pl.BlockSpec((1, wo), lambda i: (0, 0)),
        ],
        out_specs=[
            pl.BlockSpec((eb, wo), lambda i: (i, 0)),
            pl.BlockSpec((8, wo), lambda i: (0, 0)),
            pl.BlockSpec((8, wo), lambda i: (0, 0)),
        ],
        out_shape=[
            jax.ShapeDtypeStruct((e, wo), out_dtype),
            jax.ShapeDtypeStruct((8, wo), jnp.float32),
            jax.ShapeDtypeStruct((8, wo), jnp.float32),
        ],
    )(z, a, c, w, b)
    return zn, s8.sum(0), q8.sum(0)


def _edge_final_body(z_ref, a_ref, c_ref, w_ref, b_ref, out_ref):
    h = jnp.maximum(z_ref[...] * a_ref[...] + c_ref[...], 0.0)
    o = jnp.dot(h, w_ref[...], preferred_element_type=jnp.float32) + b_ref[...]
    out_ref[...] = jax.nn.sigmoid(o)


def _edge_final_body_1d(z_ref, a_ref, c_ref, w_ref, b_ref, out_ref):
    h = jnp.maximum(z_ref[...] * a_ref[...] + c_ref[...], 0.0)
    o = jnp.dot(h, w_ref[...], preferred_element_type=jnp.float32)
    o = o[:, 0:1] + b_ref[...]
    out_ref[...] = jax.nn.sigmoid(o).reshape(1, -1)


def _edge_final(z, a, c, w2, b2, eb, out1d=False):
    e, wi = z.shape
    grid = (e // eb,)
    if out1d:
        w2p = jnp.pad(w2, ((0, 0), (0, 127)))
        return pl.pallas_call(
            _edge_final_body_1d,
            grid=grid,
            in_specs=[
                pl.BlockSpec((eb, wi), lambda i: (i, 0)),
                pl.BlockSpec((1, wi), lambda i: (0, 0)),
                pl.BlockSpec((1, wi), lambda i: (0, 0)),
                pl.BlockSpec((wi, 128), lambda i: (0, 0)),
                pl.BlockSpec((1, 1), lambda i: (0, 0)),
            ],
            out_specs=pl.BlockSpec((1, eb), lambda i: (0, i)),
            out_shape=jax.ShapeDtypeStruct((1, e), jnp.float32),
        )(z, a, c, w2p, b2)
    return pl.pallas_call(
        _edge_final_body,
        grid=grid,
        in_specs=[
            pl.BlockSpec((eb, wi), lambda i: (i, 0)),
            pl.BlockSpec((1, wi), lambda i: (0, 0)),
            pl.BlockSpec((1, wi), lambda i: (0, 0)),
            pl.BlockSpec((wi, 1), lambda i: (0, 0)),
            pl.BlockSpec((1, 1), lambda i: (0, 0)),
        ],
        out_specs=pl.BlockSpec((eb, 1), lambda i: (i, 0)),
        out_shape=jax.ShapeDtypeStruct((e, 1), jnp.float32),
    )(z, a, c, w2, b2)


def _proj1_body(x_ref, wa_ref, wb_ref, b_ref, t_ref):
    x = x_ref[...]
    wb = wb_ref[...]
    u = jnp.dot(x, wb, preferred_element_type=jnp.float32)
    p = jnp.dot(x, wa_ref[...] - wb, preferred_element_type=jnp.float32) + b_ref[...]
    t_ref[...] = jnp.concatenate([u, p], axis=1)


def _proj1(x, w1, b1):
    n, d = x.shape
    wo = w1.shape[1]
    wa, wb = w1[:d], w1[d:]
    return pl.pallas_call(
        _proj1_body,
        out_shape=jax.ShapeDtypeStruct((n, 2 * wo), jnp.float32),
    )(x, wa, wb, b1.reshape(1, wo))


def _node2_body(acc_ref, x_ref, wa_ref, wb_ref, b_ref,
                h1_ref, p_ref, u_ref, cnt_ref):
    acc = acc_ref[...].sum(0)
    cnt = acc[:, 64:80].sum(axis=1, keepdims=True)
    cnt = jnp.maximum(cnt, 1.0)
    ec = acc[:, :64] / cnt
    h1 = jnp.concatenate([ec, x_ref[...]], axis=1)
    h1_ref[...] = h1
    wb = wb_ref[...]
    p_ref[...] = jnp.dot(h1, wa_ref[...] - wb, preferred_element_type=jnp.float32) + b_ref[...]
    u_ref[...] = jnp.dot(h1, wb, preferred_element_type=jnp.float32)
    cnt_ref[...] = cnt


def _node2(acc, x, w1, b1):
    n, d = x.shape
    d1 = 64 + d
    wo = w1.shape[1]
    wa, wb = w1[:d1], w1[d1:]
    return pl.pallas_call(
        _node2_body,
        out_shape=[
            jax.ShapeDtypeStruct((n, d1), jnp.float32),
            jax.ShapeDtypeStruct((n, wo), jnp.float32),
            jax.ShapeDtypeStruct((n, wo), jnp.float32),
            jax.ShapeDtypeStruct((n, 1), jnp.float32),
        ],
    )(acc, x, wa, wb, b1.reshape(1, wo))


def _node3a_body(acc_ref, cnt_ref, h1_ref, wn1_ref, bn1_ref, we_ref,
                 h2_ref, zn_ref, q_ref, s_ref, q2_ref):
    i = pl.program_id(0)
    cnt = jnp.maximum(cnt_ref[...], 1.0)
    ec = acc_ref[...].sum(0) / cnt
    h2 = jnp.concatenate([ec, h1_ref[...]], axis=1)
    h2_ref[...] = h2
    zn = jnp.dot(h2, wn1_ref[...], preferred_element_type=jnp.float32) + bn1_ref[...]
    zn_ref[...] = zn
    q_ref[...] = jnp.dot(h2, we_ref[...], preferred_element_type=jnp.float32)
    s8 = zn.reshape(-1, 8, zn.shape[-1]).sum(0)
    q8 = (zn * zn).reshape(-1, 8, zn.shape[-1]).sum(0)

    @pl.when(i == 0)
    def _():
        s_ref[...] = s8
        q2_ref[...] = q8

    @pl.when(i > 0)
    def _():
        s_ref[...] += s8
        q2_ref[...] += q8


def _node3a(acc, cnt, h1, nh, we1, nb=2000):
    n, d1 = h1.shape
    w = acc.shape[-1]
    d2 = w + d1
    wq = we1.shape[1]
    grid = (n // nb,)
    h2, zn, q, s8, q8 = pl.pallas_call(
        _node3a_body,
        grid=grid,
        in_specs=[
            pl.BlockSpec((_NC, nb, w), lambda i: (0, i, 0)),
            pl.BlockSpec((nb, 1), lambda i: (i, 0)),
            pl.BlockSpec((nb, d1), lambda i: (i, 0)),
            pl.BlockSpec((d2, 256), lambda i: (0, 0)),
            pl.BlockSpec((1, 256), lambda i: (0, 0)),
            pl.BlockSpec((d2, wq), lambda i: (0, 0)),
        ],
        out_specs=[
            pl.BlockSpec((nb, d2), lambda i: (i, 0)),
            pl.BlockSpec((nb, 256), lambda i: (i, 0)),
            pl.BlockSpec((nb, wq), lambda i: (i, 0)),
            pl.BlockSpec((8, 256), lambda i: (0, 0)),
            pl.BlockSpec((8, 256), lambda i: (0, 0)),
        ],
        out_shape=[
            jax.ShapeDtypeStruct((n, d2), jnp.float32),
            jax.ShapeDtypeStruct((n, 256), jnp.float32),
            jax.ShapeDtypeStruct((n, wq), jnp.float32),
            jax.ShapeDtypeStruct((8, 256), jnp.float32),
            jax.ShapeDtypeStruct((8, 256), jnp.float32),
        ],
    )(acc, cnt, h1, nh["l1"]["w"], nh["l1"]["b"].reshape(1, -1), we1)
    return h2, zn, q, s8.sum(0), q8.sum(0)


_NC = 2
_NS = 16
_NW = _NC * _NS
_L = 16
_C = 80


def _sc_gather_combine_call(ta, tb, src, dst, sign, oa, ob, wout, pack=False,
                            out_bf16=False):
    n, w = ta.shape
    e = src.shape[0]
    cc = 40 if w > 128 else _C
    nch = (e // _NW) // cc
    src3 = src.reshape(_NW, nch, cc)
    dst3 = dst.reshape(_NW, nch, cc)
    f_n = wout // _L
    zrows, zw = (cc // 2, 2 * wout) if pack else (cc, wout)
    zshape = (e // 2, 2 * wout) if pack else (e, wout)
    zdt = jnp.bfloat16 if out_bf16 else jnp.float32
    mesh = plsc.VectorSubcoreMesh(core_axis_name="c", subcore_axis_name="s")

    @functools.partial(
        pl.kernel,
        out_type=[jax.ShapeDtypeStruct(zshape, zdt),
                  jax.ShapeDtypeStruct((_NW, 2, wout), jnp.float32)],
        mesh=mesh,
        compiler_params=pltpu.CompilerParams(use_tc_tiling_on_sc=True,
                                             needs_layout_passes=not out_bf16),
        scratch_types=[
            pltpu.VMEM((nch, cc), jnp.int32),
            pltpu.VMEM((nch, cc), jnp.int32),
            pltpu.VMEM((2, cc, w), jnp.float32),
            pltpu.VMEM((2, cc, w), jnp.float32),
            pltpu.VMEM((2, zrows, zw), zdt),
            pltpu.VMEM((2, wout), jnp.float32),
            pltpu.SemaphoreType.DMA,
            pltpu.SemaphoreType.DMA,
            pltpu.SemaphoreType.DMA,
            pltpu.SemaphoreType.DMA,
            pltpu.SemaphoreType.DMA,
            pltpu.SemaphoreType.DMA,
        ])
    def k(ta_h, tb_h, s3_h, d3_h, z_h, st_h, si_v, di_v, a_v, b_v, z_v, st_v,
          ga0, ga1, gb0, gb1, zs0, zs1):
        wid = lax.axis_index("s") * _NC + lax.axis_index("c")
        row0 = wid * nch
        gsem = (ga0, ga1)
        bsem = (gb0, gb1)
        zsem = (zs0, zs1)
        pltpu.sync_copy(s3_h.at[wid], si_v)
        pltpu.sync_copy(d3_h.at[wid], di_v)
        zero = jnp.zeros((_L,), jnp.float32)
        for f in range(f_n):
            st_v[0, pl.ds(f * _L, _L)] = zero
            st_v[1, pl.ds(f * _L, _L)] = zero

        def fire(t, buf):
            pltpu.async_copy(ta_h.at[si_v.at[t]], a_v.at[buf], gsem[buf])
            pltpu.async_copy(tb_h.at[di_v.at[t]], b_v.at[buf], bsem[buf])

        def gwait(t, buf):
            pltpu.make_async_copy(ta_h.at[si_v.at[t]], a_v.at[buf],
                                  gsem[buf]).wait()
            pltpu.make_async_copy(tb_h.at[di_v.at[t]], b_v.at[buf],
                                  bsem[buf]).wait()

        def zdrain(t, buf):
            pltpu.make_async_copy(
                z_v.at[buf], z_h.at[pl.ds((row0 + t) * zrows, zrows), :],
                zsem[buf]).wait()

        npk = 2 if pack else 1

        def process(t, buf):
            gwait(t, buf)

            @pl.when(t >= 2)
            def _():
                zdrain(t, buf)

            for f0 in range(0, f_n, 8):
                fg = range(f0, min(f0 + 8, f_n))

                def mkrow(j, fg=fg):
                    def row(r, rc):
                        ns, nq = list(rc[0]), list(rc[1])
                        zvs = []
                        for i, f in enumerate(fg):
                            av = a_v[buf, j * zrows + r, pl.ds(oa + f * _L, _L)]
                            bv = b_v[buf, j * zrows + r, pl.ds(ob + f * _L, _L)]
                            zv = av + bv if sign > 0 else av - bv
                            if out_bf16:
                                zvs.append(zv)
                            else:
                                z_v[buf, r, pl.ds(j * wout + f * _L, _L)] = zv
                            ns[i] = ns[i] + zv
                            nq[i] = nq[i] + zv * zv
                        if out_bf16:
                            for kk in range(0, len(fg), 2):
                                pk = plsc.pack(zvs[kk], zvs[kk + 1],
                                               format=plsc.PackFormat.INTERLEAVED)
                                z_v[buf, r,
                                    pl.ds(j * wout + fg[kk] * _L, 2 * _L)] = pk
                        return (tuple(ns), tuple(nq))
                    return row

                z0 = (tuple(zero for _ in fg), tuple(zero for _ in fg))
                for j in range(npk):
                    z0 = lax.fori_loop(0, zrows, mkrow(j), z0)
                s_l, q_l = z0
                for i, f in enumerate(fg):
                    st_v[0, pl.ds(f * _L, _L)] += s_l[i]
                    st_v[1, pl.ds(f * _L, _L)] += q_l[i]
            pltpu.async_copy(
                z_v.at[buf], z_h.at[pl.ds((row0 + t) * zrows, zrows), :],
                zsem[buf])

            @pl.when(t + 2 < nch)
            def _():
                fire(t + 2, buf)

        fire(0, 0)
        fire(1, 1)

        def pair(i, carry):
            process(2 * i, 0)
            process(2 * i + 1, 1)
            return carry

        lax.fori_loop(0, nch // 2, pair, 0)
        if nch % 2:
            process(nch - 1, 0)
        zdrain(nch - 2, nch % 2)
        zdrain(nch - 1, 1 - nch % 2)
        pltpu.sync_copy(st_v, st_h.at[wid])

    return k(ta, tb, src3, dst3)


def _sc_scatter_call(msg, dst, n, a, c, pack=False):
    mrows, mw = msg.shape
    e = 2 * mrows if pack else mrows
    wr = mw // 2 if pack else mw
    wt = 128
    nch = (e // _NW) // _C
    crows = _C // 2 if pack else _C
    dst3 = dst.reshape(_NW, nch, _C)
    f_n = wr // _L
    ncr = n // _C
    nzj = -(-ncr // _NS)
    mesh = plsc.VectorSubcoreMesh(core_axis_name="c", subcore_axis_name="s")

    @functools.partial(
        pl.kernel,
        out_type=jax.ShapeDtypeStruct((_NC, n, wt), jnp.float32),
        mesh=mesh,
        compiler_params=pltpu.CompilerParams(use_tc_tiling_on_sc=True),
        scratch_types=[
            pltpu.VMEM((4, _C), jnp.int32),
            pltpu.VMEM((2, crows, mw), jnp.float32),
            pltpu.VMEM((2, _C, wt), jnp.float32),
            pltpu.VMEM((1, wr), jnp.float32),
            pltpu.VMEM((1, wr), jnp.float32),
            pltpu.VMEM_SHARED((n, wt), jnp.float32),
            pltpu.SemaphoreType.DMA,
            pltpu.SemaphoreType.DMA,
            pltpu.SemaphoreType.DMA,
            pltpu.SemaphoreType.DMA,
            pltpu.SemaphoreType.DMA,
            pltpu.SemaphoreType.DMA,
            pltpu.SemaphoreType.DMA,
            pltpu.SemaphoreType.DMA,
        ])
    def k(msg_h, d3_h, a_h, c_h, acc_h, di_v, zb_v, m_v, av_v, cv_v, table,
          ls0, ls1, ss0, ss1, is0, is1, is2, is3):
        cid = lax.axis_index("c")
        sid = lax.axis_index("s")
        wid = sid * _NC + cid
        row0 = wid * nch
        lsem = (ls0, ls1)
        ssem = (ss0, ss1)
        isem = (is0, is1, is2, is3)
        pltpu.sync_copy(a_h, av_v)
        pltpu.sync_copy(c_h, cv_v)
        zero = jnp.zeros((_L,), jnp.float32)

        def zrow(r, carry):
            for f in range(wt // _L):
                m_v[0, r, pl.ds(f * _L, _L)] = zero
                m_v[1, r, pl.ds(f * _L, _L)] = zero
            return carry

        lax.fori_loop(0, _C, zrow, 0)
        for j in range(nzj):
            row = pl.multiple_of((sid + j * _NS) * _C, _C)

            @pl.when(row < n)
            def _():
                pltpu.sync_copy(m_v.at[0], table.at[pl.ds(row, _C), :])

        if wr < wt:
            def crow(r, carry):
                m_v[0, r, pl.ds(wr, _L)] = jnp.full((_L,), 1.0 / _L,
                                                    jnp.float32)
                m_v[1, r, pl.ds(wr, _L)] = jnp.full((_L,), 1.0 / _L,
                                                    jnp.float32)
                return carry

            lax.fori_loop(0, _C, crow, 0)
        plsc.subcore_barrier()

        av_l = [av_v[0, pl.ds(f * _L, _L)] for f in range(f_n)]
        cv_l = [cv_v[0, pl.ds(f * _L, _L)] for f in range(f_n)]

        def fire(t, buf, islot):
            pltpu.async_copy(msg_h.at[pl.ds((row0 + t) * crows, crows), :],
                             zb_v.at[buf], lsem[buf])
            pltpu.async_copy(d3_h.at[wid, pl.ds(t, 1), :],
                             di_v.at[pl.ds(islot, 1), :], isem[islot])

        def lwait(t, buf, islot):
            pltpu.make_async_copy(msg_h.at[pl.ds((row0 + t) * crows, crows), :],
                                  zb_v.at[buf], lsem[buf]).wait()
            pltpu.make_async_copy(d3_h.at[wid, pl.ds(t, 1), :],
                                  di_v.at[pl.ds(islot, 1), :],
                                  isem[islot]).wait()

        def sdrain(buf, islot):
            pltpu.make_async_copy(m_v.at[buf], table.at[di_v.at[islot]],
                                  ssem[buf]).wait()

        npk = 2 if pack else 1

        def process(t, buf, islot):
            lwait(t, buf, islot)

            @pl.when(t >= 2)
            def _():
                sdrain(buf, (islot + 2) % 4)

            for j in range(npk):
                def row(r, carry, j=j):
                    for f in range(f_n):
                        zv = zb_v[buf, r, pl.ds(j * wr + f * _L, _L)]
                        m_v[buf, j * crows + r, pl.ds(f * _L, _L)] = jnp.maximum(
                            zv * av_l[f] + cv_l[f], 0.0)
                    return carry

                lax.fori_loop(0, crows, row, 0)
            pltpu.async_copy(m_v.at[buf], table.at[di_v.at[islot]],
                             ssem[buf], add=True)

            @pl.when(t + 2 < nch)
            def _():
                fire(t + 2, buf, (islot + 2) % 4)

        fire(0, 0, 0)
        fire(1, 1, 1)

        def quad(i, carry):
            process(4 * i, 0, 0)
            process(4 * i + 1, 1, 1)
            process(4 * i + 2, 0, 2)
            process(4 * i + 3, 1, 3)
            return carry

        lax.fori_loop(0, nch // 4, quad, 0)
        for j in range(nch % 4):
            t = nch - nch % 4 + j
            process(t, t % 2, t % 4)
        sdrain((nch - 2) % 2, (nch - 2) % 4)
        sdrain((nch - 1) % 2, (nch - 1) % 4)
        plsc.subcore_barrier()

        for j in range(nzj):
            row = pl.multiple_of((sid + j * _NS) * _C, _C)

            @pl.when(row < n)
            def _():
                pltpu.sync_copy(table.at[pl.ds(row, _C), :], m_v.at[0])
                pltpu.sync_copy(m_v.at[0], acc_h.at[cid, pl.ds(row, _C), :])

    return k(msg, dst3, a, c)


def _gather_combine(t1, src, dst, wout):
    z, st = _sc_gather_combine_call(t1, t1, src, dst, 1, 0, wout, wout)
    return z, st[:, 0, :].sum(0), st[:, 1, :].sum(0)


def _gather_diff(q, src, dst):
    z, st = _sc_gather_combine_call(q, q, src, dst, -1, 0, 0, q.shape[1])
    return z, st[:, 0, :].sum(0), st[:, 1, :].sum(0)


def _ilv(v):
    x = v.reshape(-1, 2, 16)
    return jnp.stack([x[:, 0], x[:, 1]], axis=-1).reshape(v.shape)


def _bn_affine(bn, m, v):
    s = bn["g"] / jnp.sqrt(v + EPS)
    return (s.reshape(1, -1), (bn["b"] - m * s).reshape(1, -1))


def _edge_conv(blocks, ta, tb, oa, ob, wout, src, dst, n, pack=False):
    e = src.shape[0]

    def pk_vec(v):
        return jnp.concatenate([v, v], axis=1) if pack else v

    def pk_mat(wm):
        if not pack:
            return wm
        wz = jnp.zeros_like(wm)
        return jnp.concatenate(
            [jnp.concatenate([wm, wz], 1), jnp.concatenate([wz, wm], 1)], 0)

    def unpk(s):
        return s[:wout] + s[wout:] if pack else s

    z1, st = _sc_gather_combine_call(ta, tb, src, dst, 1, oa, ob, wout, pack)
    s1, q1 = st[:, 0, :].sum(0), st[:, 1, :].sum(0)
    m1 = s1 / e
    a1, c1 = _bn_affine(blocks[0]["bn"], m1, q1 / e - m1 * m1)
    z2, s2, q2 = _mlp_step(z1, pk_vec(a1), pk_vec(c1),
                           pk_mat(blocks[1]["lin"]["w"]),
                           pk_vec(blocks[1]["lin"]["b"].reshape(1, -1)),
                           out_dtype=jnp.bfloat16)
    s2, q2 = unpk(s2), unpk(q2)
    m2 = s2 / e
    a2, c2 = _bn_affine(blocks[1]["bn"], m2, q2 / e - m2 * m2)
    z3, s3, q3 = _mlp_step(z2, pk_vec(a2), pk_vec(c2),
                           pk_mat(blocks[2]["lin"]["w"]),
                           pk_vec(blocks[2]["lin"]["b"].reshape(1, -1)))
    s3, q3 = unpk(s3), unpk(q3)
    m3 = s3 / e
    a3, c3 = _bn_affine(blocks[2]["bn"], m3, q3 / e - m3 * m3)
    return _sc_scatter_call(z3, dst, n, a3, c3, pack)


def kernel(x, edge_index, params):
    n = x.shape[0]
    e = edge_index.shape[1]
    src = edge_index[0]
    dst = edge_index[1]

    t1 = _proj1(x, params["ec1"][0]["lin"]["w"], params["ec1"][0]["lin"]["b"])
    acc1 = _edge_conv(params["ec1"], t1, t1, 0, 64, 64, src, dst, n, pack=True)
    h1, p2, u2, cnt = _node2(acc1, x, params["ec2"][0]["lin"]["w"],
                             params["ec2"][0]["lin"]["b"])
    acc2 = _edge_conv(params["ec2"], u2, p2, 0, 0, 128, src, dst, n)
    nh = params["node_head"]
    h2, zn, q, sn, qn = _node3a(acc2, cnt, h1, nh, params["edge_head"]["l1"]["w"])
    mn = sn / n
    an, cn = _bn_affine(nh["bn"], mn, qn / n - mn * mn)
    node_out = _edge_final(zn, an, cn, nh["l2"]["w"], nh["l2"]["b"].reshape(1, 1),
                           2000)

    ze, se, qe = _gather_diff(q, src, dst)
    me = se / e
    ae, ce = _bn_affine(params["edge_head"]["bn"], me, qe / e - me * me)
    edge_out = _edge_final(ze, ae, ce, params["edge_head"]["l2"]["w"],
                           params["edge_head"]["l2"]["b"].reshape(1, 1),
                           6400, out1d=True)
    return (node_out, edge_out.reshape(e, 1))


# speedup vs baseline: 1.4875x; 1.0001x over previous
import functools
import jax
import jax.numpy as jnp
from jax import lax
from jax.experimental import pallas as pl
from jax.experimental.pallas import tpu as pltpu
from jax.experimental.pallas import tpu_sc as plsc

EPS = 1e-5
EB = 8000


def _mlp_step_body(z_ref, a_ref, c_ref, w_ref, b_ref, out_ref, s_ref, q_ref):
    i = pl.program_id(0)
    z = z_ref[...].astype(jnp.float32)
    h = jnp.maximum(z * a_ref[...] + c_ref[...], 0.0)
    zn = jnp.dot(h, w_ref[...], preferred_element_type=jnp.float32) + b_ref[...]
    out_ref[...] = zn.astype(out_ref.dtype)
    zn = zn.astype(out_ref.dtype).astype(jnp.float32)
    s8 = zn.reshape(-1, 8, zn.shape[-1]).sum(0)
    q8 = (zn * zn).reshape(-1, 8, zn.shape[-1]).sum(0)

    @pl.when(i == 0)
    def _():
        s_ref[...] = s8
        q_ref[...] = q8

    @pl.when(i > 0)
    def _():
        s_ref[...] += s8
        q_ref[...] += q8


def _mlp_step(z, a, c, w, b, out_dtype=jnp.float32):
    e, _ = z.shape
    wo = w.shape[1]
    eb = EB
    grid = (e // eb,)
    zn, s8, q8 = pl.pallas_call(
        _mlp_step_body,
        grid=grid,
        in_specs=[
            pl.BlockSpec((eb, z.shape[1]), lambda i: (i, 0)),
            pl.BlockSpec((1, z.shape[1]), lambda i: (0, 0)),
            pl.BlockSpec((1, z.shape[1]), lambda i: (0, 0)),
            pl.BlockSpec(w.shape, lambda i: (0, 0)),
            pl.BlockSpec((1, wo), lambda i: (0, 0)),
        ],
        out_specs=[
            pl.BlockSpec((eb, wo), lambda i: (i, 0)),
            pl.BlockSpec((8, wo), lambda i: (0, 0)),
            pl.BlockSpec((8, wo), lambda i: (0, 0)),
        ],
        out_shape=[
            jax.ShapeDtypeStruct((e, wo), out_dtype),
            jax.ShapeDtypeStruct((8, wo), jnp.float32),
            jax.ShapeDtypeStruct((8, wo), jnp.float32),
        ],
    )(z, a, c, w, b)
    return zn, s8.sum(0), q8.sum(0)


def _edge_final_body(z_ref, a_ref, c_ref, w_ref, b_ref, out_ref):
    h = jnp.maximum(z_ref[...] * a_ref[...] + c_ref[...], 0.0)
    o = jnp.dot(h, w_ref[...], preferred_element_type=jnp.float32) + b_ref[...]
    out_ref[...] = jax.nn.sigmoid(o)


def _edge_final_body_1d(z_ref, a_ref, c_ref, w_ref, b_ref, out_ref):
    h = jnp.maximum(z_ref[...] * a_ref[...] + c_ref[...], 0.0)
    o = jnp.dot(h, w_ref[...], preferred_element_type=jnp.float32)
    o = o[:, 0:1] + b_ref[...]
    out_ref[...] = jax.nn.sigmoid(o).reshape(1, -1)


def _edge_final(z, a, c, w2, b2, eb, out1d=False):
    e, wi = z.shape
    grid = (e // eb,)
    if out1d:
        w2p = jnp.pad(w2, ((0, 0), (0, 127)))
        return pl.pallas_call(
            _edge_final_body_1d,
            grid=grid,
            in_specs=[
                pl.BlockSpec((eb, wi), lambda i: (i, 0)),
                pl.BlockSpec((1, wi), lambda i: (0, 0)),
                pl.BlockSpec((1, wi), lambda i: (0, 0)),
                pl.BlockSpec((wi, 128), lambda i: (0, 0)),
                pl.BlockSpec((1, 1), lambda i: (0, 0)),
            ],
            out_specs=pl.BlockSpec((1, eb), lambda i: (0, i)),
            out_shape=jax.ShapeDtypeStruct((1, e), jnp.float32),
        )(z, a, c, w2p, b2)
    return pl.pallas_call(
        _edge_final_body,
        grid=grid,
        in_specs=[
            pl.BlockSpec((eb, wi), lambda i: (i, 0)),
            pl.BlockSpec((1, wi), lambda i: (0, 0)),
            pl.BlockSpec((1, wi), lambda i: (0, 0)),
            pl.BlockSpec((wi, 1), lambda i: (0, 0)),
            pl.BlockSpec((1, 1), lambda i: (0, 0)),
        ],
        out_specs=pl.BlockSpec((eb, 1), lambda i: (i, 0)),
        out_shape=jax.ShapeDtypeStruct((e, 1), jnp.float32),
    )(z, a, c, w2, b2)


def _proj1_body(x_ref, wa_ref, wb_ref, b_ref, t_ref):
    x = x_ref[...]
    wb = wb_ref[...]
    u = jnp.dot(x, wb, preferred_element_type=jnp.float32)
    p = jnp.dot(x, wa_ref[...] - wb, preferred_element_type=jnp.float32) + b_ref[...]
    t_ref[...] = jnp.concatenate([u, p], axis=1)


def _proj1(x, w1, b1):
    n, d = x.shape
    wo = w1.shape[1]
    wa, wb = w1[:d], w1[d:]
    return pl.pallas_call(
        _proj1_body,
        out_shape=jax.ShapeDtypeStruct((n, 2 * wo), jnp.float32),
    )(x, wa, wb, b1.reshape(1, wo))


def _node2_body(acc_ref, x_ref, wa_ref, wb_ref, b_ref,
                h1_ref, p_ref, u_ref, cnt_ref):
    acc = acc_ref[...].sum(0)
    cnt = acc[:, 64:80].sum(axis=1, keepdims=True)
    cnt = jnp.maximum(cnt, 1.0)
    ec = acc[:, :64] / cnt
    h1 = jnp.concatenate([ec, x_ref[...]], axis=1)
    h1_ref[...] = h1
    wb = wb_ref[...]
    p_ref[...] = jnp.dot(h1, wa_ref[...] - wb, preferred_element_type=jnp.float32) + b_ref[...]
    u_ref[...] = jnp.dot(h1, wb, preferred_element_type=jnp.float32)
    cnt_ref[...] = cnt


def _node2(acc, x, w1, b1):
    n, d = x.shape
    d1 = 64 + d
    wo = w1.shape[1]
    wa, wb = w1[:d1], w1[d1:]
    return pl.pallas_call(
        _node2_body,
        out_shape=[
            jax.ShapeDtypeStruct((n, d1), jnp.float32),
            jax.ShapeDtypeStruct((n, wo), jnp.float32),
            jax.ShapeDtypeStruct((n, wo), jnp.float32),
            jax.ShapeDtypeStruct((n, 1), jnp.float32),
        ],
    )(acc, x, wa, wb, b1.reshape(1, wo))


def _node3a_body(acc_ref, cnt_ref, h1_ref, wn1_ref, bn1_ref, we_ref,
                 h2_ref, zn_ref, q_ref, s_ref, q2_ref):
    i = pl.program_id(0)
    cnt = jnp.maximum(cnt_ref[...], 1.0)
    ec = acc_ref[...].sum(0) / cnt
    h2 = jnp.concatenate([ec, h1_ref[...]], axis=1)
    h2_ref[...] = h2
    zn = jnp.dot(h2, wn1_ref[...], preferred_element_type=jnp.float32) + bn1_ref[...]
    zn_ref[...] = zn
    q_ref[...] = jnp.dot(h2, we_ref[...], preferred_element_type=jnp.float32)
    s8 = zn.reshape(-1, 8, zn.shape[-1]).sum(0)
    q8 = (zn * zn).reshape(-1, 8, zn.shape[-1]).sum(0)

    @pl.when(i == 0)
    def _():
        s_ref[...] = s8
        q2_ref[...] = q8

    @pl.when(i > 0)
    def _():
        s_ref[...] += s8
        q2_ref[...] += q8


def _node3a(acc, cnt, h1, nh, we1, nb=2000):
    n, d1 = h1.shape
    w = acc.shape[-1]
    d2 = w + d1
    wq = we1.shape[1]
    grid = (n // nb,)
    h2, zn, q, s8, q8 = pl.pallas_call(
        _node3a_body,
        grid=grid,
        in_specs=[
            pl.BlockSpec((_NC, nb, w), lambda i: (0, i, 0)),
            pl.BlockSpec((nb, 1), lambda i: (i, 0)),
            pl.BlockSpec((nb, d1), lambda i: (i, 0)),
            pl.BlockSpec((d2, 256), lambda i: (0, 0)),
            pl.BlockSpec((1, 256), lambda i: (0, 0)),
            pl.BlockSpec((d2, wq), lambda i: (0, 0)),
        ],
        out_specs=[
            pl.BlockSpec((nb, d2), lambda i: (i, 0)),
            pl.BlockSpec((nb, 256), lambda i: (i, 0)),
            pl.BlockSpec((nb, wq), lambda i: (i, 0)),
            pl.BlockSpec((8, 256), lambda i: (0, 0)),
            pl.BlockSpec((8, 256), lambda i: (0, 0)),
        ],
        out_shape=[
            jax.ShapeDtypeStruct((n, d2), jnp.float32),
            jax.ShapeDtypeStruct((n, 256), jnp.float32),
            jax.ShapeDtypeStruct((n, wq), jnp.float32),
            jax.ShapeDtypeStruct((8, 256), jnp.float32),
            jax.ShapeDtypeStruct((8, 256), jnp.float32),
        ],
    )(acc, cnt, h1, nh["l1"]["w"], nh["l1"]["b"].reshape(1, -1), we1)
    return h2, zn, q, s8.sum(0), q8.sum(0)


_NC = 2
_NS = 16
_NW = _NC * _NS
_L = 16
_C = 80


def _sc_gather_combine_call(ta, tb, src, dst, sign, oa, ob, wout, pack=False,
                            out_bf16=False):
    n, w = ta.shape
    e = src.shape[0]
    cc = 40 if w > 128 else _C
    nch = (e // _NW) // cc
    src3 = src.reshape(_NW, nch, cc)
    dst3 = dst.reshape(_NW, nch, cc)
    f_n = wout // _L
    zrows, zw = (cc // 2, 2 * wout) if pack else (cc, wout)
    zshape = (e // 2, 2 * wout) if pack else (e, wout)
    zdt = jnp.bfloat16 if out_bf16 else jnp.float32
    mesh = plsc.VectorSubcoreMesh(core_axis_name="c", subcore_axis_name="s")

    @functools.partial(
        pl.kernel,
        out_type=[jax.ShapeDtypeStruct(zshape, zdt),
                  jax.ShapeDtypeStruct((_NW, 2, wout), jnp.float32)],
        mesh=mesh,
        compiler_params=pltpu.CompilerParams(use_tc_tiling_on_sc=True,
                                             needs_layout_passes=not out_bf16),
        scratch_types=[
            pltpu.VMEM((nch, cc), jnp.int32),
            pltpu.VMEM((nch, cc), jnp.int32),
            pltpu.VMEM((2, cc, w), jnp.float32),
            pltpu.VMEM((2, cc, w), jnp.float32),
            pltpu.VMEM((2, zrows, zw), zdt),
            pltpu.VMEM((2, wout), jnp.float32),
            pltpu.SemaphoreType.DMA,
            pltpu.SemaphoreType.DMA,
            pltpu.SemaphoreType.DMA,
            pltpu.SemaphoreType.DMA,
            pltpu.SemaphoreType.DMA,
            pltpu.SemaphoreType.DMA,
        ])
    def k(ta_h, tb_h, s3_h, d3_h, z_h, st_h, si_v, di_v, a_v, b_v, z_v, st_v,
          ga0, ga1, gb0, gb1, zs0, zs1):
        wid = lax.axis_index("s") * _NC + lax.axis_index("c")
        row0 = wid * nch
        gsem = (ga0, ga1)
        bsem = (gb0, gb1)
        zsem = (zs0, zs1)
        pltpu.sync_copy(s3_h.at[wid], si_v)
        pltpu.sync_copy(d3_h.at[wid], di_v)
        zero = jnp.zeros((_L,), jnp.float32)
        for f in range(f_n):
            st_v[0, pl.ds(f * _L, _L)] = zero
            st_v[1, pl.ds(f * _L, _L)] = zero

        def fire(t, buf):
            pltpu.async_copy(ta_h.at[si_v.at[t]], a_v.at[buf], gsem[buf])
            pltpu.async_copy(tb_h.at[di_v.at[t]], b_v.at[buf], bsem[buf])

        def gwait(t, buf):
            pltpu.make_async_copy(ta_h.at[si_v.at[t]], a_v.at[buf],
                                  gsem[buf]).wait()
            pltpu.make_async_copy(tb_h.at[di_v.at[t]], b_v.at[buf],
                                  bsem[buf]).wait()

        def zdrain(t, buf):
            pltpu.make_async_copy(
                z_v.at[buf], z_h.at[pl.ds((row0 + t) * zrows, zrows), :],
                zsem[buf]).wait()

        npk = 2 if pack else 1

        def process(t, buf):
            gwait(t, buf)

            @pl.when(t >= 2)
            def _():
                zdrain(t, buf)

            for f0 in range(0, f_n, 8):
                fg = range(f0, min(f0 + 8, f_n))

                def mkrow(j, fg=fg):
                    def row(r, rc):
                        ns, nq = list(rc[0]), list(rc[1])
                        zvs = []
                        for i, f in enumerate(fg):
                            av = a_v[buf, j * zrows + r, pl.ds(oa + f * _L, _L)]
                            bv = b_v[buf, j * zrows + r, pl.ds(ob + f * _L, _L)]
                            zv = av + bv if sign > 0 else av - bv
                            if out_bf16:
                                zvs.append(zv)
                            else:
                                z_v[buf, r, pl.ds(j * wout + f * _L, _L)] = zv
                            ns[i] = ns[i] + zv
                            nq[i] = nq[i] + zv * zv
                        if out_bf16:
                            for kk in range(0, len(fg), 2):
                                pk = plsc.pack(zvs[kk], zvs[kk + 1],
                                               format=plsc.PackFormat.INTERLEAVED)
                                z_v[buf, r,
                                    pl.ds(j * wout + fg[kk] * _L, 2 * _L)] = pk
                        return (tuple(ns), tuple(nq))
                    return row

                z0 = (tuple(zero for _ in fg), tuple(zero for _ in fg))
                for j in range(npk):
                    z0 = lax.fori_loop(0, zrows, mkrow(j), z0)
                s_l, q_l = z0
                for i, f in enumerate(fg):
                    st_v[0, pl.ds(f * _L, _L)] += s_l[i]
                    st_v[1, pl.ds(f * _L, _L)] += q_l[i]
            pltpu.async_copy(
                z_v.at[buf], z_h.at[pl.ds((row0 + t) * zrows, zrows), :],
                zsem[buf])

            @pl.when(t + 2 < nch)
            def _():
                fire(t + 2, buf)

        fire(0, 0)
        fire(1, 1)

        def pair(i, carry):
            process(2 * i, 0)
            process(2 * i + 1, 1)
            return carry

        lax.fori_loop(0, nch // 2, pair, 0)
        if nch % 2:
            process(nch - 1, 0)
        zdrain(nch - 2, nch % 2)
        zdrain(nch - 1, 1 - nch % 2)
        pltpu.sync_copy(st_v, st_h.at[wid])

    return k(ta, tb, src3, dst3)


def _sc_scatter_call(msg, dst, n, a, c, pack=False):
    mrows, mw = msg.shape
    e = 2 * mrows if pack else mrows
    wr = mw // 2 if pack else mw
    wt = 128
    nch = (e // _NW) // _C
    crows = _C // 2 if pack else _C
    dst3 = dst.reshape(_NW, nch, _C)
    f_n = wr // _L
    ncr = n // _C
    nzj = -(-ncr // _NS)
    mesh = plsc.VectorSubcoreMesh(core_axis_name="c", subcore_axis_name="s")

    @functools.partial(
        pl.kernel,
        out_type=jax.ShapeDtypeStruct((_NC, n, wt), jnp.float32),
        mesh=mesh,
        compiler_params=pltpu.CompilerParams(use_tc_tiling_on_sc=True),
        scratch_types=[
            pltpu.VMEM((4, _C), jnp.int32),
            pltpu.VMEM((2, crows, mw), jnp.float32),
            pltpu.VMEM((2, _C, wt), jnp.float32),
            pltpu.VMEM((1, wr), jnp.float32),
            pltpu.VMEM((1, wr), jnp.float32),
            pltpu.VMEM_SHARED((n, wt), jnp.float32),
            pltpu.SemaphoreType.DMA,
            pltpu.SemaphoreType.DMA,
            pltpu.SemaphoreType.DMA,
            pltpu.SemaphoreType.DMA,
            pltpu.SemaphoreType.DMA,
            pltpu.SemaphoreType.DMA,
            pltpu.SemaphoreType.DMA,
            pltpu.SemaphoreType.DMA,
        ])
    def k(msg_h, d3_h, a_h, c_h, acc_h, di_v, zb_v, m_v, av_v, cv_v, table,
          ls0, ls1, ss0, ss1, is0, is1, is2, is3):
        cid = lax.axis_index("c")
        sid = lax.axis_index("s")
        wid = sid * _NC + cid
        row0 = wid * nch
        lsem = (ls0, ls1)
        ssem = (ss0, ss1)
        isem = (is0, is1, is2, is3)
        pltpu.sync_copy(a_h, av_v)
        pltpu.sync_copy(c_h, cv_v)
        zero = jnp.zeros((_L,), jnp.float32)

        def zrow(r, carry):
            for f in range(wt // _L):
                m_v[0, r, pl.ds(f * _L, _L)] = zero
                m_v[1, r, pl.ds(f * _L, _L)] = zero
            return carry

        lax.fori_loop(0, _C, zrow, 0)
        for j in range(nzj):
            row = pl.multiple_of((sid + j * _NS) * _C, _C)

            @pl.when(row < n)
            def _():
                pltpu.sync_copy(m_v.at[0], table.at[pl.ds(row, _C), :])

        if wr < wt:
            def crow(r, carry):
                m_v[0, r, pl.ds(wr, _L)] = jnp.full((_L,), 1.0 / _L,
                                                    jnp.float32)
                m_v[1, r, pl.ds(wr, _L)] = jnp.full((_L,), 1.0 / _L,
                                                    jnp.float32)
                return carry

            lax.fori_loop(0, _C, crow, 0)
        plsc.subcore_barrier()

        av_l = [av_v[0, pl.ds(f * _L, _L)] for f in range(f_n)]
        cv_l = [cv_v[0, pl.ds(f * _L, _L)] for f in range(f_n)]

        def fire(t, buf, islot):
            pltpu.async_copy(msg_h.at[pl.ds((row0 + t) * crows, crows), :],
                             zb_v.at[buf], lsem[buf])
            pltpu.async_copy(d3_h.at[wid, pl.ds(t, 1), :],
                             di_v.at[pl.ds(islot, 1), :], isem[islot])

        def lwait(t, buf, islot):
            pltpu.make_async_copy(msg_h.at[pl.ds((row0 + t) * crows, crows), :],
                                  zb_v.at[buf], lsem[buf]).wait()
            pltpu.make_async_copy(d3_h.at[wid, pl.ds(t, 1), :],
                                  di_v.at[pl.ds(islot, 1), :],
                                  isem[islot]).wait()

        def sdrain(buf, islot):
            pltpu.make_async_copy(m_v.at[buf], table.at[di_v.at[islot]],
                                  ssem[buf]).wait()

        npk = 2 if pack else 1

        def process(t, buf, islot):
            lwait(t, buf, islot)

            @pl.when(t >= 2)
            def _():
                sdrain(buf, (islot + 2) % 4)

            for j in range(npk):
                def row(r, carry, j=j):
                    for f in range(f_n):
                        zv = zb_v[buf, r, pl.ds(j * wr + f * _L, _L)]
                        m_v[buf, j * crows + r, pl.ds(f * _L, _L)] = jnp.maximum(
                            zv * av_l[f] + cv_l[f], 0.0)
                    return carry

                lax.fori_loop(0, crows, row, 0)
            pltpu.async_copy(m_v.at[buf], table.at[di_v.at[islot]],
                             ssem[buf], add=True)

            @pl.when(t + 2 < nch)
            def _():
                fire(t + 2, buf, (islot + 2) % 4)

        fire(0, 0, 0)
        fire(1, 1, 1)

        def quad(i, carry):
            process(4 * i, 0, 0)
            process(4 * i + 1, 1, 1)
            process(4 * i + 2, 0, 2)
            process(4 * i + 3, 1, 3)
            return carry

        lax.fori_loop(0, nch // 4, quad, 0)
        for j in range(nch % 4):
            t = nch - nch % 4 + j
            process(t, t % 2, t % 4)
        sdrain((nch - 2) % 2, (nch - 2) % 4)
        sdrain((nch - 1) % 2, (nch - 1) % 4)
        plsc.subcore_barrier()

        for j in range(nzj):
            row = pl.multiple_of((sid + j * _NS) * _C, _C)

            @pl.when(row < n)
            def _():
                pltpu.sync_copy(table.at[pl.ds(row, _C), :], m_v.at[0])
                pltpu.sync_copy(m_v.at[0], acc_h.at[cid, pl.ds(row, _C), :])

    return k(msg, dst3, a, c)


def _gather_diff(q, src, dst):
    z, st = _sc_gather_combine_call(q, q, src, dst, -1, 0, 0, q.shape[1])
    return z, st[:, 0, :].sum(0), st[:, 1, :].sum(0)


def _bn_affine(bn, m, v):
    s = bn["g"] / jnp.sqrt(v + EPS)
    return (s.reshape(1, -1), (bn["b"] - m * s).reshape(1, -1))


def _edge_conv(blocks, ta, tb, oa, ob, wout, src, dst, n, pack=False):
    e = src.shape[0]

    def pk_vec(v):
        return jnp.concatenate([v, v], axis=1) if pack else v

    def pk_mat(wm):
        if not pack:
            return wm
        wz = jnp.zeros_like(wm)
        return jnp.concatenate(
            [jnp.concatenate([wm, wz], 1), jnp.concatenate([wz, wm], 1)], 0)

    def unpk(s):
        return s[:wout] + s[wout:] if pack else s

    z1, st = _sc_gather_combine_call(ta, tb, src, dst, 1, oa, ob, wout, pack)
    s1, q1 = st[:, 0, :].sum(0), st[:, 1, :].sum(0)
    m1 = s1 / e
    a1, c1 = _bn_affine(blocks[0]["bn"], m1, q1 / e - m1 * m1)
    z2, s2, q2 = _mlp_step(z1, pk_vec(a1), pk_vec(c1),
                           pk_mat(blocks[1]["lin"]["w"]),
                           pk_vec(blocks[1]["lin"]["b"].reshape(1, -1)),
                           out_dtype=jnp.bfloat16)
    s2, q2 = unpk(s2), unpk(q2)
    m2 = s2 / e
    a2, c2 = _bn_affine(blocks[1]["bn"], m2, q2 / e - m2 * m2)
    z3, s3, q3 = _mlp_step(z2, pk_vec(a2), pk_vec(c2),
                           pk_mat(blocks[2]["lin"]["w"]),
                           pk_vec(blocks[2]["lin"]["b"].reshape(1, -1)))
    s3, q3 = unpk(s3), unpk(q3)
    m3 = s3 / e
    a3, c3 = _bn_affine(blocks[2]["bn"], m3, q3 / e - m3 * m3)
    return _sc_scatter_call(z3, dst, n, a3, c3, pack)


def kernel(x, edge_index, params):
    n = x.shape[0]
    e = edge_index.shape[1]
    src = edge_index[0]
    dst = edge_index[1]

    t1 = _proj1(x, params["ec1"][0]["lin"]["w"], params["ec1"][0]["lin"]["b"])
    acc1 = _edge_conv(params["ec1"], t1, t1, 0, 64, 64, src, dst, n, pack=True)
    h1, p2, u2, cnt = _node2(acc1, x, params["ec2"][0]["lin"]["w"],
                             params["ec2"][0]["lin"]["b"])
    acc2 = _edge_conv(params["ec2"], u2, p2, 0, 0, 128, src, dst, n)
    nh = params["node_head"]
    h2, zn, q, sn, qn = _node3a(acc2, cnt, h1, nh, params["edge_head"]["l1"]["w"])
    mn = sn / n
    an, cn = _bn_affine(nh["bn"], mn, qn / n - mn * mn)
    node_out = _edge_final(zn, an, cn, nh["l2"]["w"], nh["l2"]["b"].reshape(1, 1),
                           2000)

    ze, se, qe = _gather_diff(q, src, dst)
    me = se / e
    ae, ce = _bn_affine(params["edge_head"]["bn"], me, qe / e - me * me)
    edge_out = _edge_final(ze, ae, ce, params["edge_head"]["l2"]["w"],
                           params["edge_head"]["l2"]["b"].reshape(1, 1),
                           6400, out1d=True)
    return (node_out, edge_out.reshape(e, 1))
